# Initial kernel scaffold; baseline (speedup 1.0000x reference)
import jax
import jax.numpy as jnp
from jax.experimental import pallas as pl


def kernel(x, edge_index, Wr0, Wn0, b0, Wr1, Wn1, b1, Wr2, Wn2, b2):
    raise NotImplementedError("write your pallas kernel here")


# speedup vs baseline: 4.6110x; 4.6110x over previous
import functools

import jax
import jax.numpy as jnp
from jax import lax
from jax.experimental import pallas as pl
from jax.experimental.pallas import tpu as pltpu
from jax.experimental.pallas import tpu_sc as plsc

N_NODES = 10000
D_FEAT = 128
E_EDGES = 320000

NC, NS, L = 2, 16, 16
NW = NC * NS
EPW = E_EDGES // NW
CHUNK = 80
NCHUNK = EPW // CHUNK
RPT = 624
TAIL = N_NODES - NS * RPT
ZROWS = 16

_MESH = plsc.VectorSubcoreMesh(core_axis_name="c", subcore_axis_name="s")


def _fill(buf, rows, vec):
    def row(i, _):
        for j in range(D_FEAT // L):
            buf[i, pl.ds(j * L, L)] = vec
        return 0

    lax.fori_loop(0, rows, row, 0)


def _zero_acc(s, zbuf, acc_sh):
    def zcopy(t, _):
        pltpu.sync_copy(zbuf, acc_sh.at[pl.ds(s * RPT + t * ZROWS, ZROWS)])
        return 0

    lax.fori_loop(0, RPT // ZROWS, zcopy, 0)

    @pl.when(s == 0)
    def _():
        pltpu.sync_copy(zbuf, acc_sh.at[pl.ds(NS * RPT, TAIL)])


def _write_out(c, s, zbuf, acc_sh, out_hbm):
    def ocopy(t, _):
        off = s * RPT + t * ZROWS
        pltpu.sync_copy(acc_sh.at[pl.ds(off, ZROWS)], zbuf)
        pltpu.sync_copy(zbuf, out_hbm.at[c, pl.ds(off, ZROWS)])
        return 0

    lax.fori_loop(0, RPT // ZROWS, ocopy, 0)

    @pl.when(s == 0)
    def _():
        pltpu.sync_copy(acc_sh.at[pl.ds(NS * RPT, TAIL)], zbuf)
        pltpu.sync_copy(zbuf, out_hbm.at[c, pl.ds(NS * RPT, TAIL)])


def _sc_agg_body(h_hbm, src_hbm, dst_hbm, agg_hbm,
                 src_v, dst_v, rows_v, zbuf, agg_sh, sem):
    c = lax.axis_index("c")
    s = lax.axis_index("s")
    wid = s * NC + c

    _fill(zbuf, ZROWS, jnp.zeros((L,), jnp.float32))
    _zero_acc(s, zbuf, agg_sh)
    plsc.subcore_barrier()

    base = wid * EPW

    def step(k, _):
        off = base + k * CHUNK
        pltpu.sync_copy(src_hbm.at[pl.ds(off, CHUNK)], src_v)
        pltpu.sync_copy(dst_hbm.at[pl.ds(off, CHUNK)], dst_v)
        pltpu.async_copy(h_hbm.at[src_v], rows_v, sem).wait()
        pltpu.sync_copy(rows_v, agg_sh.at[dst_v], add=True)
        return 0

    lax.fori_loop(0, NCHUNK, step, 0)
    plsc.subcore_barrier()
    _write_out(c, s, zbuf, agg_sh, agg_hbm)


_sc_agg = pl.kernel(
    _sc_agg_body,
    out_type=jax.ShapeDtypeStruct((NC, N_NODES, D_FEAT), jnp.float32),
    mesh=_MESH,
    scratch_types=[
        pltpu.VMEM((CHUNK,), jnp.int32),
        pltpu.VMEM((CHUNK,), jnp.int32),
        pltpu.VMEM((CHUNK, D_FEAT), jnp.float32),
        pltpu.VMEM((ZROWS, D_FEAT), jnp.float32),
        pltpu.VMEM_SHARED((N_NODES, D_FEAT), jnp.float32),
        pltpu.SemaphoreType.DMA,
    ],
)


def _sc_deg_body(dst_hbm, deg_hbm, dst_v, ones_v, zbuf, deg_sh):
    c = lax.axis_index("c")
    s = lax.axis_index("s")
    wid = s * NC + c

    _fill(zbuf, ZROWS, jnp.zeros((L,), jnp.float32))
    _fill(ones_v, CHUNK, jnp.ones((L,), jnp.float32))
    _zero_acc(s, zbuf, deg_sh)
    plsc.subcore_barrier()

    base = wid * EPW

    def step(k, _):
        off = base + k * CHUNK
        pltpu.sync_copy(dst_hbm.at[pl.ds(off, CHUNK)], dst_v)
        pltpu.sync_copy(ones_v, deg_sh.at[dst_v], add=True)
        return 0

    lax.fori_loop(0, NCHUNK, step, 0)
    plsc.subcore_barrier()
    _write_out(c, s, zbuf, deg_sh, deg_hbm)


_sc_deg = pl.kernel(
    _sc_deg_body,
    out_type=jax.ShapeDtypeStruct((NC, N_NODES, D_FEAT), jnp.float32),
    mesh=_MESH,
    scratch_types=[
        pltpu.VMEM((CHUNK,), jnp.int32),
        pltpu.VMEM((CHUNK, D_FEAT), jnp.float32),
        pltpu.VMEM((ZROWS, D_FEAT), jnp.float32),
        pltpu.VMEM_SHARED((N_NODES, D_FEAT), jnp.float32),
    ],
)

ROWS_TC = 1000


def _tc_layer(h, a0, a1, d0, d1, Wr, Wn, b, act):
    def body(h_ref, a0_ref, a1_ref, d0_ref, d1_ref, wr_ref, wn_ref, b_ref,
             o_ref):
        deg = d0_ref[:, :1] + d1_ref[:, :1]
        mean = (a0_ref[...] + a1_ref[...]) / jnp.maximum(deg, 1.0)
        acc = jnp.dot(h_ref[...], wr_ref[...],
                      preferred_element_type=jnp.float32)
        acc = acc + jnp.dot(mean, wn_ref[...],
                            preferred_element_type=jnp.float32)
        acc = acc + b_ref[...]
        if act:
            acc = jnp.maximum(acc, 0.0)
        o_ref[...] = acc

    return pl.pallas_call(
        body,
        grid=(N_NODES // ROWS_TC,),
        in_specs=[
            pl.BlockSpec((ROWS_TC, D_FEAT), lambda i: (i, 0)),
            pl.BlockSpec((ROWS_TC, D_FEAT), lambda i: (i, 0)),
            pl.BlockSpec((ROWS_TC, D_FEAT), lambda i: (i, 0)),
            pl.BlockSpec((ROWS_TC, D_FEAT), lambda i: (i, 0)),
            pl.BlockSpec((ROWS_TC, D_FEAT), lambda i: (i, 0)),
            pl.BlockSpec((D_FEAT, D_FEAT), lambda i: (0, 0)),
            pl.BlockSpec((D_FEAT, D_FEAT), lambda i: (0, 0)),
            pl.BlockSpec((1, D_FEAT), lambda i: (0, 0)),
        ],
        out_specs=pl.BlockSpec((ROWS_TC, D_FEAT), lambda i: (i, 0)),
        out_shape=jax.ShapeDtypeStruct((N_NODES, D_FEAT), jnp.float32),
    )(h, a0, a1, d0, d1, Wr, Wn, b)


def kernel(x, edge_index, Wr0, Wn0, b0, Wr1, Wn1, b1, Wr2, Wn2, b2):
    src = edge_index[0]
    dst = edge_index[1]

    deg = _sc_deg(dst)
    d0, d1 = deg[0], deg[1]
    agg = _sc_agg(x, src, dst)
    h = _tc_layer(x, agg[0], agg[1], d0, d1, Wr0, Wn0,
                  b0.reshape(1, D_FEAT), True)
    agg = _sc_agg(h, src, dst)
    h = _tc_layer(h, agg[0], agg[1], d0, d1, Wr1, Wn1,
                  b1.reshape(1, D_FEAT), True)
    agg = _sc_agg(h, src, dst)
    h = _tc_layer(h, agg[0], agg[1], d0, d1, Wr2, Wn2,
                  b2.reshape(1, D_FEAT), False)
    return h


# speedup vs baseline: 8.5180x; 1.8473x over previous
import functools

import jax
import jax.numpy as jnp
from jax import lax
from jax.experimental import pallas as pl
from jax.experimental.pallas import tpu as pltpu
from jax.experimental.pallas import tpu_sc as plsc

N_NODES = 10000
D_FEAT = 128
E_EDGES = 320000

NC, NS, L = 2, 16, 16
NW = NC * NS
EPW = E_EDGES // NW
CHUNK = 80
NCHUNK = EPW // CHUNK
NPAIR = (NCHUNK - 1) // 2
RPT = 624
TAIL = N_NODES - NS * RPT
ZROWS = 16

_MESH = plsc.VectorSubcoreMesh(core_axis_name="c", subcore_axis_name="s")


def _fill(buf, rows, vec):
    def row(i, _):
        for j in range(D_FEAT // L):
            buf[i, pl.ds(j * L, L)] = vec
        return 0

    lax.fori_loop(0, rows, row, 0)


def _zero_acc(s, zbuf, acc_sh, zsems):
    def zcopy(t, _):
        @pl.when(t >= 2)
        def _():
            off0 = s * RPT + (t - 2) * ZROWS
            pltpu.make_async_copy(zbuf, acc_sh.at[pl.ds(off0, ZROWS)],
                                  zsems.at[lax.rem(t, 2)]).wait()

        off = s * RPT + t * ZROWS
        pltpu.async_copy(zbuf, acc_sh.at[pl.ds(off, ZROWS)],
                         zsems.at[lax.rem(t, 2)])
        return 0

    nz = RPT // ZROWS
    lax.fori_loop(0, nz, zcopy, 0)
    for t in (nz - 2, nz - 1):
        pltpu.make_async_copy(zbuf, acc_sh.at[pl.ds(s * RPT + t * ZROWS,
                                                    ZROWS)],
                              zsems.at[t % 2]).wait()

    @pl.when(s == 0)
    def _():
        pltpu.sync_copy(zbuf, acc_sh.at[pl.ds(NS * RPT, TAIL)])


def _write_out(c, s, zbuf, acc_sh, out_hbm, zsems):
    def ocopy(t, _):
        off = s * RPT + t * ZROWS

        @pl.when(t >= 2)
        def _():
            off0 = s * RPT + (t - 2) * ZROWS
            pltpu.make_async_copy(zbuf, out_hbm.at[c, pl.ds(off0, ZROWS)],
                                  zsems.at[lax.rem(t, 2)]).wait()

        pltpu.sync_copy(acc_sh.at[pl.ds(off, ZROWS)], zbuf)
        pltpu.async_copy(zbuf, out_hbm.at[c, pl.ds(off, ZROWS)],
                         zsems.at[lax.rem(t, 2)])
        return 0

    nz = RPT // ZROWS
    lax.fori_loop(0, nz, ocopy, 0)
    for t in (nz - 2, nz - 1):
        pltpu.make_async_copy(zbuf, out_hbm.at[c, pl.ds(s * RPT + t * ZROWS,
                                                        ZROWS)],
                              zsems.at[t % 2]).wait()

    @pl.when(s == 0)
    def _():
        pltpu.sync_copy(acc_sh.at[pl.ds(NS * RPT, TAIL)], zbuf)
        pltpu.sync_copy(zbuf, out_hbm.at[c, pl.ds(NS * RPT, TAIL)])


def _sc_agg_body(h_hbm, src_hbm, dst_hbm, agg_hbm,
                 src_v, dst_v, rows0, rows1, zbuf, agg_sh,
                 g0, g1, s0, s1, zsems):
    c = lax.axis_index("c")
    s = lax.axis_index("s")
    wid = s * NC + c

    pltpu.async_copy(src_hbm.at[wid], src_v, g0).wait()
    pltpu.async_copy(dst_hbm.at[wid], dst_v, g1).wait()
    _fill(zbuf, ZROWS, jnp.zeros((L,), jnp.float32))
    _zero_acc(s, zbuf, agg_sh, zsems)
    plsc.subcore_barrier()

    def fire_g(k, rows, sem):
        pltpu.async_copy(h_hbm.at[src_v.at[pl.ds(k * CHUNK, CHUNK)]],
                         rows, sem)

    def wait_g(k, rows, sem):
        pltpu.make_async_copy(h_hbm.at[src_v.at[pl.ds(k * CHUNK, CHUNK)]],
                              rows, sem).wait()

    def fire_s(k, rows, sem):
        pltpu.async_copy(rows, agg_sh.at[dst_v.at[k]], sem, add=True)

    def wait_s(k, rows, sem):
        pltpu.make_async_copy(rows, agg_sh.at[dst_v.at[k]], sem).wait()

    fire_g(0, rows0, g0)

    def pair(i, _):
        a = 2 * i
        b = a + 1

        @pl.when(i > 0)
        def _():
            wait_s(lax.max(b - 2, 0), rows1, s1)

        fire_g(b, rows1, g1)
        wait_g(a, rows0, g0)
        fire_s(a, rows0, s0)
        wait_g(b, rows1, g1)
        fire_s(b, rows1, s1)
        wait_s(a, rows0, s0)
        fire_g(a + 2, rows0, g0)
        return 0

    lax.fori_loop(0, NPAIR, pair, 0)
    last = NCHUNK - 1
    wait_g(last, rows0, g0)
    fire_s(last, rows0, s0)
    wait_s(last - 1, rows1, s1)
    wait_s(last, rows0, s0)

    plsc.subcore_barrier()
    _write_out(c, s, zbuf, agg_sh, agg_hbm, zsems)


_sc_agg_raw = pl.kernel(
    _sc_agg_body,
    out_type=jax.ShapeDtypeStruct((NC, N_NODES, D_FEAT), jnp.float32),
    mesh=_MESH,
    scratch_types=[
        pltpu.VMEM((EPW,), jnp.int32),
        pltpu.VMEM((NCHUNK, CHUNK), jnp.int32),
        pltpu.VMEM((CHUNK, D_FEAT), jnp.float32),
        pltpu.VMEM((CHUNK, D_FEAT), jnp.float32),
        pltpu.VMEM((ZROWS, D_FEAT), jnp.float32),
        pltpu.VMEM_SHARED((N_NODES, D_FEAT), jnp.float32),
        pltpu.SemaphoreType.DMA,
        pltpu.SemaphoreType.DMA,
        pltpu.SemaphoreType.DMA,
        pltpu.SemaphoreType.DMA,
        pltpu.SemaphoreType.DMA((2,)),
    ],
)


def _sc_agg(h, src3, dst3):
    return _sc_agg_raw(h, src3, dst3)


def _sc_deg_body(dst_hbm, deg_hbm, dst_v, ones_v, zbuf, deg_sh,
                 s0, s1, zsems):
    c = lax.axis_index("c")
    s = lax.axis_index("s")
    wid = s * NC + c

    pltpu.async_copy(dst_hbm.at[wid], dst_v, s0).wait()
    _fill(zbuf, ZROWS, jnp.zeros((L,), jnp.float32))
    _fill(ones_v, CHUNK, jnp.ones((L,), jnp.float32))
    _zero_acc(s, zbuf, deg_sh, zsems)
    plsc.subcore_barrier()

    def step(k, _):
        @pl.when(k >= 2)
        def _():
            pltpu.make_async_copy(
                ones_v, deg_sh.at[dst_v.at[lax.max(k - 2, 0)]],
                zsems.at[lax.rem(k, 2)]).wait()

        pltpu.async_copy(ones_v, deg_sh.at[dst_v.at[k]],
                         zsems.at[lax.rem(k, 2)], add=True)
        return 0

    lax.fori_loop(0, NCHUNK, step, 0)
    for k in (NCHUNK - 2, NCHUNK - 1):
        pltpu.make_async_copy(ones_v, deg_sh.at[dst_v.at[k]],
                              zsems.at[k % 2]).wait()

    plsc.subcore_barrier()
    _write_out(c, s, zbuf, deg_sh, deg_hbm, zsems)


_sc_deg_raw = pl.kernel(
    _sc_deg_body,
    out_type=jax.ShapeDtypeStruct((NC, N_NODES, D_FEAT), jnp.float32),
    mesh=_MESH,
    scratch_types=[
        pltpu.VMEM((NCHUNK, CHUNK), jnp.int32),
        pltpu.VMEM((CHUNK, D_FEAT), jnp.float32),
        pltpu.VMEM((ZROWS, D_FEAT), jnp.float32),
        pltpu.VMEM_SHARED((N_NODES, D_FEAT), jnp.float32),
        pltpu.SemaphoreType.DMA,
        pltpu.SemaphoreType.DMA,
        pltpu.SemaphoreType.DMA((2,)),
    ],
)

ROWS_TC = 1000


def _tc_layer(h, a0, a1, d0, d1, Wr, Wn, b, act):
    def body(h_ref, a0_ref, a1_ref, d0_ref, d1_ref, wr_ref, wn_ref, b_ref,
             o_ref):
        deg = d0_ref[:, :1] + d1_ref[:, :1]
        mean = (a0_ref[...] + a1_ref[...]) / jnp.maximum(deg, 1.0)
        acc = jnp.dot(h_ref[...], wr_ref[...],
                      preferred_element_type=jnp.float32)
        acc = acc + jnp.dot(mean, wn_ref[...],
                            preferred_element_type=jnp.float32)
        acc = acc + b_ref[...]
        if act:
            acc = jnp.maximum(acc, 0.0)
        o_ref[...] = acc

    return pl.pallas_call(
        body,
        grid=(N_NODES // ROWS_TC,),
        in_specs=[
            pl.BlockSpec((ROWS_TC, D_FEAT), lambda i: (i, 0)),
            pl.BlockSpec((ROWS_TC, D_FEAT), lambda i: (i, 0)),
            pl.BlockSpec((ROWS_TC, D_FEAT), lambda i: (i, 0)),
            pl.BlockSpec((ROWS_TC, D_FEAT), lambda i: (i, 0)),
            pl.BlockSpec((ROWS_TC, D_FEAT), lambda i: (i, 0)),
            pl.BlockSpec((D_FEAT, D_FEAT), lambda i: (0, 0)),
            pl.BlockSpec((D_FEAT, D_FEAT), lambda i: (0, 0)),
            pl.BlockSpec((1, D_FEAT), lambda i: (0, 0)),
        ],
        out_specs=pl.BlockSpec((ROWS_TC, D_FEAT), lambda i: (i, 0)),
        out_shape=jax.ShapeDtypeStruct((N_NODES, D_FEAT), jnp.float32),
    )(h, a0, a1, d0, d1, Wr, Wn, b)


def kernel(x, edge_index, Wr0, Wn0, b0, Wr1, Wn1, b1, Wr2, Wn2, b2):
    src3 = edge_index[0].reshape(NW, EPW)
    dst3 = edge_index[1].reshape(NW, NCHUNK, CHUNK)

    deg = _sc_deg_raw(dst3)
    d0, d1 = deg[0], deg[1]
    agg = _sc_agg(x, src3, dst3)
    h = _tc_layer(x, agg[0], agg[1], d0, d1, Wr0, Wn0,
                  b0.reshape(1, D_FEAT), True)
    agg = _sc_agg(h, src3, dst3)
    h = _tc_layer(h, agg[0], agg[1], d0, d1, Wr1, Wn1,
                  b1.reshape(1, D_FEAT), True)
    agg = _sc_agg(h, src3, dst3)
    h = _tc_layer(h, agg[0], agg[1], d0, d1, Wr2, Wn2,
                  b2.reshape(1, D_FEAT), False)
    return h


# speedup vs baseline: 9.2508x; 1.0860x over previous
import functools

import jax
import jax.numpy as jnp
from jax import lax
from jax.experimental import pallas as pl
from jax.experimental.pallas import tpu as pltpu
from jax.experimental.pallas import tpu_sc as plsc

N_NODES = 10000
D_FEAT = 128
E_EDGES = 320000

NC, NS, L = 2, 16, 16
NW = NC * NS
EPW = E_EDGES // NW
CHUNK = 80
NCHUNK = EPW // CHUNK
NPAIR = (NCHUNK - 1) // 2
RPT = 624
TAIL = N_NODES - NS * RPT
ZROWS = 16

_MESH = plsc.VectorSubcoreMesh(core_axis_name="c", subcore_axis_name="s")


def _fill(buf, rows, vec):
    def row(i, _):
        for j in range(D_FEAT // L):
            buf[i, pl.ds(j * L, L)] = vec
        return 0

    lax.fori_loop(0, rows, row, 0)


def _zero_acc(s, zbuf, acc_sh, zsems):
    def zcopy(t, _):
        @pl.when(t >= 2)
        def _():
            off0 = s * RPT + (t - 2) * ZROWS
            pltpu.make_async_copy(zbuf, acc_sh.at[pl.ds(off0, ZROWS)],
                                  zsems.at[lax.rem(t, 2)]).wait()

        off = s * RPT + t * ZROWS
        pltpu.async_copy(zbuf, acc_sh.at[pl.ds(off, ZROWS)],
                         zsems.at[lax.rem(t, 2)])
        return 0

    nz = RPT // ZROWS
    lax.fori_loop(0, nz, zcopy, 0)
    for t in (nz - 2, nz - 1):
        pltpu.make_async_copy(zbuf, acc_sh.at[pl.ds(s * RPT + t * ZROWS,
                                                    ZROWS)],
                              zsems.at[t % 2]).wait()

    @pl.when(s == 0)
    def _():
        pltpu.sync_copy(zbuf, acc_sh.at[pl.ds(NS * RPT, TAIL)])


def _write_out(c, s, zbuf, acc_sh, out_hbm, zsems):
    def ocopy(t, _):
        off = s * RPT + t * ZROWS

        @pl.when(t >= 2)
        def _():
            off0 = s * RPT + (t - 2) * ZROWS
            pltpu.make_async_copy(zbuf, out_hbm.at[c, pl.ds(off0, ZROWS)],
                                  zsems.at[lax.rem(t, 2)]).wait()

        pltpu.sync_copy(acc_sh.at[pl.ds(off, ZROWS)], zbuf)
        pltpu.async_copy(zbuf, out_hbm.at[c, pl.ds(off, ZROWS)],
                         zsems.at[lax.rem(t, 2)])
        return 0

    nz = RPT // ZROWS
    lax.fori_loop(0, nz, ocopy, 0)
    for t in (nz - 2, nz - 1):
        pltpu.make_async_copy(zbuf, out_hbm.at[c, pl.ds(s * RPT + t * ZROWS,
                                                        ZROWS)],
                              zsems.at[t % 2]).wait()

    @pl.when(s == 0)
    def _():
        pltpu.sync_copy(acc_sh.at[pl.ds(NS * RPT, TAIL)], zbuf)
        pltpu.sync_copy(zbuf, out_hbm.at[c, pl.ds(NS * RPT, TAIL)])


NBUF = 4
NRING = (NCHUNK // NBUF) * NBUF


def _sc_agg_body(h_hbm, src_hbm, dst_hbm, agg_hbm,
                 srcs, dsts, rows, zbuf, agg_sh,
                 isems, gsems, ssems, zsems):
    c = lax.axis_index("c")
    s = lax.axis_index("s")
    wid = s * NC + c

    _fill(zbuf, ZROWS, jnp.zeros((L,), jnp.float32))
    _zero_acc(s, zbuf, agg_sh, zsems)
    plsc.subcore_barrier()

    base = wid * EPW

    def fire_i(k, j):
        off = base + k * CHUNK
        pltpu.async_copy(src_hbm.at[pl.ds(off, CHUNK)], srcs[j], isems[j])
        pltpu.async_copy(dst_hbm.at[pl.ds(off, CHUNK)], dsts[j], isems[j])

    def wait_i(k, j):
        off = base + k * CHUNK
        pltpu.make_async_copy(src_hbm.at[pl.ds(off, CHUNK)], srcs[j],
                              isems[j]).wait()
        pltpu.make_async_copy(dst_hbm.at[pl.ds(off, CHUNK)], dsts[j],
                              isems[j]).wait()

    def fire_g(j):
        pltpu.async_copy(h_hbm.at[srcs[j]], rows[j], gsems[j])

    def wait_g(j):
        pltpu.make_async_copy(h_hbm.at[srcs[j]], rows[j], gsems[j]).wait()

    def fire_s(j):
        pltpu.async_copy(rows[j], agg_sh.at[dsts[j]], ssems[j], add=True)

    def wait_s(j):
        pltpu.make_async_copy(rows[j], agg_sh.at[dsts[j]], ssems[j]).wait()

    def ring(i, _):
        for j in range(NBUF):
            k = NBUF * i + j

            @pl.when(i > 0)
            def _():
                wait_s(j)

            fire_i(k, j)
        for j in range(NBUF):
            wait_i(NBUF * i + j, j)
            fire_g(j)
        for j in range(NBUF):
            wait_g(j)
            fire_s(j)
        return 0

    lax.fori_loop(0, NRING // NBUF, ring, 0)
    for j in range(NBUF):
        wait_s(j)
    for k in range(NRING, NCHUNK):
        j = k - NRING
        fire_i(k, j)
        wait_i(k, j)
        fire_g(j)
        wait_g(j)
        fire_s(j)
        wait_s(j)

    plsc.subcore_barrier()
    _write_out(c, s, zbuf, agg_sh, agg_hbm, zsems)


_sc_agg_raw = pl.kernel(
    _sc_agg_body,
    out_type=jax.ShapeDtypeStruct((NC, N_NODES, D_FEAT), jnp.float32),
    mesh=_MESH,
    scratch_types=[
        [pltpu.VMEM((CHUNK,), jnp.int32) for _ in range(NBUF)],
        [pltpu.VMEM((CHUNK,), jnp.int32) for _ in range(NBUF)],
        [pltpu.VMEM((CHUNK, D_FEAT), jnp.float32) for _ in range(NBUF)],
        pltpu.VMEM((ZROWS, D_FEAT), jnp.float32),
        pltpu.VMEM_SHARED((N_NODES, D_FEAT), jnp.float32),
        [pltpu.SemaphoreType.DMA for _ in range(NBUF)],
        [pltpu.SemaphoreType.DMA for _ in range(NBUF)],
        [pltpu.SemaphoreType.DMA for _ in range(NBUF)],
        pltpu.SemaphoreType.DMA((2,)),
    ],
)


def _sc_agg(h, src3, dst3):
    return _sc_agg_raw(h, src3, dst3)


def _sc_deg_body(dst_hbm, deg_hbm, dst_v, ones_v, zbuf, deg_sh,
                 s0, s1, zsems):
    c = lax.axis_index("c")
    s = lax.axis_index("s")
    wid = s * NC + c

    pltpu.async_copy(dst_hbm.at[wid], dst_v, s0).wait()
    _fill(zbuf, ZROWS, jnp.zeros((L,), jnp.float32))
    _fill(ones_v, CHUNK, jnp.ones((L,), jnp.float32))
    _zero_acc(s, zbuf, deg_sh, zsems)
    plsc.subcore_barrier()

    def step(k, _):
        @pl.when(k >= 2)
        def _():
            pltpu.make_async_copy(
                ones_v, deg_sh.at[dst_v.at[lax.max(k - 2, 0)]],
                zsems.at[lax.rem(k, 2)]).wait()

        pltpu.async_copy(ones_v, deg_sh.at[dst_v.at[k]],
                         zsems.at[lax.rem(k, 2)], add=True)
        return 0

    lax.fori_loop(0, NCHUNK, step, 0)
    for k in (NCHUNK - 2, NCHUNK - 1):
        pltpu.make_async_copy(ones_v, deg_sh.at[dst_v.at[k]],
                              zsems.at[k % 2]).wait()

    plsc.subcore_barrier()
    _write_out(c, s, zbuf, deg_sh, deg_hbm, zsems)


_sc_deg_raw = pl.kernel(
    _sc_deg_body,
    out_type=jax.ShapeDtypeStruct((NC, N_NODES, D_FEAT), jnp.float32),
    mesh=_MESH,
    scratch_types=[
        pltpu.VMEM((NCHUNK, CHUNK), jnp.int32),
        pltpu.VMEM((CHUNK, D_FEAT), jnp.float32),
        pltpu.VMEM((ZROWS, D_FEAT), jnp.float32),
        pltpu.VMEM_SHARED((N_NODES, D_FEAT), jnp.float32),
        pltpu.SemaphoreType.DMA,
        pltpu.SemaphoreType.DMA,
        pltpu.SemaphoreType.DMA((2,)),
    ],
)

ROWS_TC = 1000


def _tc_layer(h, a0, a1, d0, d1, Wr, Wn, b, act):
    def body(h_ref, a0_ref, a1_ref, d0_ref, d1_ref, wr_ref, wn_ref, b_ref,
             o_ref):
        deg = d0_ref[:, :1] + d1_ref[:, :1]
        mean = (a0_ref[...] + a1_ref[...]) / jnp.maximum(deg, 1.0)
        acc = jnp.dot(h_ref[...], wr_ref[...],
                      preferred_element_type=jnp.float32)
        acc = acc + jnp.dot(mean, wn_ref[...],
                            preferred_element_type=jnp.float32)
        acc = acc + b_ref[...]
        if act:
            acc = jnp.maximum(acc, 0.0)
        o_ref[...] = acc

    return pl.pallas_call(
        body,
        grid=(N_NODES // ROWS_TC,),
        in_specs=[
            pl.BlockSpec((ROWS_TC, D_FEAT), lambda i: (i, 0)),
            pl.BlockSpec((ROWS_TC, D_FEAT), lambda i: (i, 0)),
            pl.BlockSpec((ROWS_TC, D_FEAT), lambda i: (i, 0)),
            pl.BlockSpec((ROWS_TC, D_FEAT), lambda i: (i, 0)),
            pl.BlockSpec((ROWS_TC, D_FEAT), lambda i: (i, 0)),
            pl.BlockSpec((D_FEAT, D_FEAT), lambda i: (0, 0)),
            pl.BlockSpec((D_FEAT, D_FEAT), lambda i: (0, 0)),
            pl.BlockSpec((1, D_FEAT), lambda i: (0, 0)),
        ],
        out_specs=pl.BlockSpec((ROWS_TC, D_FEAT), lambda i: (i, 0)),
        out_shape=jax.ShapeDtypeStruct((N_NODES, D_FEAT), jnp.float32),
    )(h, a0, a1, d0, d1, Wr, Wn, b)


def kernel(x, edge_index, Wr0, Wn0, b0, Wr1, Wn1, b1, Wr2, Wn2, b2):
    src3 = edge_index[0]
    dst3 = edge_index[1]
    dst_deg = dst3.reshape(NW, NCHUNK, CHUNK)

    deg = _sc_deg_raw(dst_deg)
    d0, d1 = deg[0], deg[1]
    agg = _sc_agg(x, src3, dst3)
    h = _tc_layer(x, agg[0], agg[1], d0, d1, Wr0, Wn0,
                  b0.reshape(1, D_FEAT), True)
    agg = _sc_agg(h, src3, dst3)
    h = _tc_layer(h, agg[0], agg[1], d0, d1, Wr1, Wn1,
                  b1.reshape(1, D_FEAT), True)
    agg = _sc_agg(h, src3, dst3)
    h = _tc_layer(h, agg[0], agg[1], d0, d1, Wr2, Wn2,
                  b2.reshape(1, D_FEAT), False)
    return h


# speedup vs baseline: 9.3330x; 1.0089x over previous
import functools

import jax
import jax.numpy as jnp
from jax import lax
from jax.experimental import pallas as pl
from jax.experimental.pallas import tpu as pltpu
from jax.experimental.pallas import tpu_sc as plsc

N_NODES = 10000
D_FEAT = 128
E_EDGES = 320000

NC, NS, L = 2, 16, 16
NW = NC * NS
EPW = E_EDGES // NW
CHUNK = 80
NCHUNK = EPW // CHUNK
NPAIR = (NCHUNK - 1) // 2
RPT = 624
TAIL = N_NODES - NS * RPT
ZROWS = 48

_MESH = plsc.VectorSubcoreMesh(core_axis_name="c", subcore_axis_name="s")


def _fill(buf, rows, vec):
    def row(i, _):
        for j in range(D_FEAT // L):
            buf[i, pl.ds(j * L, L)] = vec
        return 0

    lax.fori_loop(0, rows, row, 0)


def _zero_acc(s, zbuf, acc_sh, zsems):
    def zcopy(t, _):
        @pl.when(t >= 2)
        def _():
            off0 = s * RPT + (t - 2) * ZROWS
            pltpu.make_async_copy(zbuf, acc_sh.at[pl.ds(off0, ZROWS)],
                                  zsems.at[lax.rem(t, 2)]).wait()

        off = s * RPT + t * ZROWS
        pltpu.async_copy(zbuf, acc_sh.at[pl.ds(off, ZROWS)],
                         zsems.at[lax.rem(t, 2)])
        return 0

    nz = RPT // ZROWS
    lax.fori_loop(0, nz, zcopy, 0)
    for t in (nz - 2, nz - 1):
        pltpu.make_async_copy(zbuf, acc_sh.at[pl.ds(s * RPT + t * ZROWS,
                                                    ZROWS)],
                              zsems.at[t % 2]).wait()

    @pl.when(s == 0)
    def _():
        pltpu.sync_copy(zbuf.at[pl.ds(0, TAIL)],
                        acc_sh.at[pl.ds(NS * RPT, TAIL)])


def _write_out(c, s, acc_sh, out_hbm, zsems):
    pltpu.async_copy(acc_sh.at[pl.ds(s * RPT, RPT)],
                     out_hbm.at[c, pl.ds(s * RPT, RPT)], zsems.at[0])

    @pl.when(s == 0)
    def _():
        pltpu.async_copy(acc_sh.at[pl.ds(NS * RPT, TAIL)],
                         out_hbm.at[c, pl.ds(NS * RPT, TAIL)], zsems.at[1])

    pltpu.make_async_copy(acc_sh.at[pl.ds(s * RPT, RPT)],
                          out_hbm.at[c, pl.ds(s * RPT, RPT)],
                          zsems.at[0]).wait()

    @pl.when(s == 0)
    def _():
        pltpu.make_async_copy(acc_sh.at[pl.ds(NS * RPT, TAIL)],
                              out_hbm.at[c, pl.ds(NS * RPT, TAIL)],
                              zsems.at[1]).wait()


NBUF = 4
NRING = (NCHUNK // NBUF) * NBUF


def _sc_agg_body(h_hbm, src_hbm, dst_hbm, agg_hbm,
                 srcs, dsts, rows, zbuf, agg_sh,
                 isems, gsems, ssems, zsems):
    c = lax.axis_index("c")
    s = lax.axis_index("s")
    wid = s * NC + c

    _fill(zbuf, ZROWS, jnp.zeros((L,), jnp.float32))
    _zero_acc(s, zbuf, agg_sh, zsems)
    plsc.subcore_barrier()

    base = wid * EPW

    def fire_i(k, j):
        off = base + k * CHUNK
        pltpu.async_copy(src_hbm.at[pl.ds(off, CHUNK)], srcs[j], isems[j])
        pltpu.async_copy(dst_hbm.at[pl.ds(off, CHUNK)], dsts[j], isems[j])

    def wait_i(k, j):
        off = base + k * CHUNK
        pltpu.make_async_copy(src_hbm.at[pl.ds(off, CHUNK)], srcs[j],
                              isems[j]).wait()
        pltpu.make_async_copy(dst_hbm.at[pl.ds(off, CHUNK)], dsts[j],
                              isems[j]).wait()

    def fire_g(j):
        pltpu.async_copy(h_hbm.at[srcs[j]], rows[j], gsems[j])

    def wait_g(j):
        pltpu.make_async_copy(h_hbm.at[srcs[j]], rows[j], gsems[j]).wait()

    def fire_s(j):
        pltpu.async_copy(rows[j], agg_sh.at[dsts[j]], ssems[j], add=True)

    def wait_s(j):
        pltpu.make_async_copy(rows[j], agg_sh.at[dsts[j]], ssems[j]).wait()

    def ring(i, _):
        for j in range(NBUF):
            k = NBUF * i + j

            @pl.when(i > 0)
            def _():
                wait_s(j)

            fire_i(k, j)
        for j in range(NBUF):
            wait_i(NBUF * i + j, j)
            fire_g(j)
        for j in range(NBUF):
            wait_g(j)
            fire_s(j)
        return 0

    lax.fori_loop(0, NRING // NBUF, ring, 0)
    for j in range(NBUF):
        wait_s(j)
    for k in range(NRING, NCHUNK):
        j = k - NRING
        fire_i(k, j)
        wait_i(k, j)
        fire_g(j)
        wait_g(j)
        fire_s(j)
        wait_s(j)

    plsc.subcore_barrier()
    _write_out(c, s, agg_sh, agg_hbm, zsems)


_sc_agg_raw = pl.kernel(
    _sc_agg_body,
    out_type=jax.ShapeDtypeStruct((NC, N_NODES, D_FEAT), jnp.float32),
    mesh=_MESH,
    scratch_types=[
        [pltpu.VMEM((CHUNK,), jnp.int32) for _ in range(NBUF)],
        [pltpu.VMEM((CHUNK,), jnp.int32) for _ in range(NBUF)],
        [pltpu.VMEM((CHUNK, D_FEAT), jnp.float32) for _ in range(NBUF)],
        pltpu.VMEM((ZROWS, D_FEAT), jnp.float32),
        pltpu.VMEM_SHARED((N_NODES, D_FEAT), jnp.float32),
        [pltpu.SemaphoreType.DMA for _ in range(NBUF)],
        [pltpu.SemaphoreType.DMA for _ in range(NBUF)],
        [pltpu.SemaphoreType.DMA for _ in range(NBUF)],
        pltpu.SemaphoreType.DMA((2,)),
    ],
)


def _sc_agg(h, src3, dst3):
    return _sc_agg_raw(h, src3, dst3)


def _sc_deg_body(dst_hbm, deg_hbm, dst_v, ones_v, zbuf, deg_sh,
                 s0, s1, zsems):
    c = lax.axis_index("c")
    s = lax.axis_index("s")
    wid = s * NC + c

    pltpu.async_copy(dst_hbm.at[wid], dst_v, s0).wait()
    _fill(zbuf, ZROWS, jnp.zeros((L,), jnp.float32))
    _fill(ones_v, CHUNK, jnp.ones((L,), jnp.float32))
    _zero_acc(s, zbuf, deg_sh, zsems)
    plsc.subcore_barrier()

    def step(k, _):
        @pl.when(k >= 2)
        def _():
            pltpu.make_async_copy(
                ones_v, deg_sh.at[dst_v.at[lax.max(k - 2, 0)]],
                zsems.at[lax.rem(k, 2)]).wait()

        pltpu.async_copy(ones_v, deg_sh.at[dst_v.at[k]],
                         zsems.at[lax.rem(k, 2)], add=True)
        return 0

    lax.fori_loop(0, NCHUNK, step, 0)
    for k in (NCHUNK - 2, NCHUNK - 1):
        pltpu.make_async_copy(ones_v, deg_sh.at[dst_v.at[k]],
                              zsems.at[k % 2]).wait()

    plsc.subcore_barrier()
    _write_out(c, s, deg_sh, deg_hbm, zsems)


_sc_deg_raw = pl.kernel(
    _sc_deg_body,
    out_type=jax.ShapeDtypeStruct((NC, N_NODES, D_FEAT), jnp.float32),
    mesh=_MESH,
    scratch_types=[
        pltpu.VMEM((NCHUNK, CHUNK), jnp.int32),
        pltpu.VMEM((CHUNK, D_FEAT), jnp.float32),
        pltpu.VMEM((ZROWS, D_FEAT), jnp.float32),
        pltpu.VMEM_SHARED((N_NODES, D_FEAT), jnp.float32),
        pltpu.SemaphoreType.DMA,
        pltpu.SemaphoreType.DMA,
        pltpu.SemaphoreType.DMA((2,)),
    ],
)

ROWS_TC = 1000


def _tc_layer(h, a0, a1, d0, d1, Wr, Wn, b, act):
    def body(h_ref, a0_ref, a1_ref, d0_ref, d1_ref, wr_ref, wn_ref, b_ref,
             o_ref):
        deg = d0_ref[:, :1] + d1_ref[:, :1]
        mean = (a0_ref[...] + a1_ref[...]) / jnp.maximum(deg, 1.0)
        acc = jnp.dot(h_ref[...], wr_ref[...],
                      preferred_element_type=jnp.float32)
        acc = acc + jnp.dot(mean, wn_ref[...],
                            preferred_element_type=jnp.float32)
        acc = acc + b_ref[...]
        if act:
            acc = jnp.maximum(acc, 0.0)
        o_ref[...] = acc

    return pl.pallas_call(
        body,
        grid=(N_NODES // ROWS_TC,),
        in_specs=[
            pl.BlockSpec((ROWS_TC, D_FEAT), lambda i: (i, 0)),
            pl.BlockSpec((ROWS_TC, D_FEAT), lambda i: (i, 0)),
            pl.BlockSpec((ROWS_TC, D_FEAT), lambda i: (i, 0)),
            pl.BlockSpec((ROWS_TC, D_FEAT), lambda i: (i, 0)),
            pl.BlockSpec((ROWS_TC, D_FEAT), lambda i: (i, 0)),
            pl.BlockSpec((D_FEAT, D_FEAT), lambda i: (0, 0)),
            pl.BlockSpec((D_FEAT, D_FEAT), lambda i: (0, 0)),
            pl.BlockSpec((1, D_FEAT), lambda i: (0, 0)),
        ],
        out_specs=pl.BlockSpec((ROWS_TC, D_FEAT), lambda i: (i, 0)),
        out_shape=jax.ShapeDtypeStruct((N_NODES, D_FEAT), jnp.float32),
    )(h, a0, a1, d0, d1, Wr, Wn, b)


def kernel(x, edge_index, Wr0, Wn0, b0, Wr1, Wn1, b1, Wr2, Wn2, b2):
    src3 = edge_index[0]
    dst3 = edge_index[1]
    dst_deg = dst3.reshape(NW, NCHUNK, CHUNK)

    deg = _sc_deg_raw(dst_deg)
    d0, d1 = deg[0], deg[1]
    agg = _sc_agg(x, src3, dst3)
    h = _tc_layer(x, agg[0], agg[1], d0, d1, Wr0, Wn0,
                  b0.reshape(1, D_FEAT), True)
    agg = _sc_agg(h, src3, dst3)
    h = _tc_layer(h, agg[0], agg[1], d0, d1, Wr1, Wn1,
                  b1.reshape(1, D_FEAT), True)
    agg = _sc_agg(h, src3, dst3)
    h = _tc_layer(h, agg[0], agg[1], d0, d1, Wr2, Wn2,
                  b2.reshape(1, D_FEAT), False)
    return h


# speedup vs baseline: 10.0451x; 1.0763x over previous
import functools

import jax
import jax.numpy as jnp
from jax import lax
from jax.experimental import pallas as pl
from jax.experimental.pallas import tpu as pltpu
from jax.experimental.pallas import tpu_sc as plsc

N_NODES = 10000
D_FEAT = 128
E_EDGES = 320000

NC, NS, L = 2, 16, 16
NW = NC * NS
EPW = E_EDGES // NW
CHUNK = 80
NCHUNK = EPW // CHUNK
NPAIR = (NCHUNK - 1) // 2
RPT = 624
TAIL = N_NODES - NS * RPT
ZROWS = 48

_MESH = plsc.VectorSubcoreMesh(core_axis_name="c", subcore_axis_name="s")


def _fill(buf, rows, vec):
    def row(i, _):
        for j in range(D_FEAT // L):
            buf[i, pl.ds(j * L, L)] = vec
        return 0

    lax.fori_loop(0, rows, row, 0)


def _zero_acc(s, zbuf, acc_sh, zsems):
    def zcopy(t, _):
        @pl.when(t >= 2)
        def _():
            off0 = s * RPT + (t - 2) * ZROWS
            pltpu.make_async_copy(zbuf, acc_sh.at[pl.ds(off0, ZROWS)],
                                  zsems.at[lax.rem(t, 2)]).wait()

        off = s * RPT + t * ZROWS
        pltpu.async_copy(zbuf, acc_sh.at[pl.ds(off, ZROWS)],
                         zsems.at[lax.rem(t, 2)])
        return 0

    nz = RPT // ZROWS
    lax.fori_loop(0, nz, zcopy, 0)
    for t in (nz - 2, nz - 1):
        pltpu.make_async_copy(zbuf, acc_sh.at[pl.ds(s * RPT + t * ZROWS,
                                                    ZROWS)],
                              zsems.at[t % 2]).wait()

    @pl.when(s == 0)
    def _():
        pltpu.sync_copy(zbuf.at[pl.ds(0, TAIL)],
                        acc_sh.at[pl.ds(NS * RPT, TAIL)])


def _write_out(c, s, acc_sh, out_hbm, zsems):
    pltpu.async_copy(acc_sh.at[pl.ds(s * RPT, RPT)],
                     out_hbm.at[c, pl.ds(s * RPT, RPT)], zsems.at[0])

    @pl.when(s == 0)
    def _():
        pltpu.async_copy(acc_sh.at[pl.ds(NS * RPT, TAIL)],
                         out_hbm.at[c, pl.ds(NS * RPT, TAIL)], zsems.at[1])

    pltpu.make_async_copy(acc_sh.at[pl.ds(s * RPT, RPT)],
                          out_hbm.at[c, pl.ds(s * RPT, RPT)],
                          zsems.at[0]).wait()

    @pl.when(s == 0)
    def _():
        pltpu.make_async_copy(acc_sh.at[pl.ds(NS * RPT, TAIL)],
                              out_hbm.at[c, pl.ds(NS * RPT, TAIL)],
                              zsems.at[1]).wait()


NBUF = 4
NRING = (NCHUNK // NBUF) * NBUF


def _sc_agg_body(h_hbm, ei_hbm, agg_hbm,
                 srcs, dsts, rows, zbuf, agg_sh,
                 isems, gsems, ssems, zsems):
    c = lax.axis_index("c")
    s = lax.axis_index("s")
    wid = s * NC + c

    _fill(zbuf, ZROWS, jnp.zeros((L,), jnp.float32))
    _zero_acc(s, zbuf, agg_sh, zsems)
    plsc.subcore_barrier()

    base = wid * EPW

    def fire_i(k, j):
        off = base + k * CHUNK
        pltpu.async_copy(ei_hbm.at[pl.ds(off, CHUNK)], srcs[j], isems[j])
        pltpu.async_copy(ei_hbm.at[pl.ds(E_EDGES + off, CHUNK)], dsts[j],
                         isems[j])

    def wait_i(k, j):
        off = base + k * CHUNK
        pltpu.make_async_copy(ei_hbm.at[pl.ds(off, CHUNK)], srcs[j],
                              isems[j]).wait()
        pltpu.make_async_copy(ei_hbm.at[pl.ds(E_EDGES + off, CHUNK)], dsts[j],
                              isems[j]).wait()

    def fire_g(j):
        pltpu.async_copy(h_hbm.at[srcs[j]], rows[j], gsems[j])

    def wait_g(j):
        pltpu.make_async_copy(h_hbm.at[srcs[j]], rows[j], gsems[j]).wait()

    def fire_s(j):
        pltpu.async_copy(rows[j], agg_sh.at[dsts[j]], ssems[j], add=True)

    def wait_s(j):
        pltpu.make_async_copy(rows[j], agg_sh.at[dsts[j]], ssems[j]).wait()

    def ring(i, _):
        for j in range(NBUF):
            k = NBUF * i + j

            @pl.when(i > 0)
            def _():
                wait_s(j)

            fire_i(k, j)
        for j in range(NBUF):
            wait_i(NBUF * i + j, j)
            fire_g(j)
        for j in range(NBUF):
            wait_g(j)
            fire_s(j)
        return 0

    lax.fori_loop(0, NRING // NBUF, ring, 0)
    for j in range(NBUF):
        wait_s(j)
    for k in range(NRING, NCHUNK):
        j = k - NRING
        fire_i(k, j)
        wait_i(k, j)
        fire_g(j)
        wait_g(j)
        fire_s(j)
        wait_s(j)

    plsc.subcore_barrier()
    _write_out(c, s, agg_sh, agg_hbm, zsems)


_sc_agg_raw = pl.kernel(
    _sc_agg_body,
    out_type=jax.ShapeDtypeStruct((NC, N_NODES, D_FEAT), jnp.float32),
    mesh=_MESH,
    scratch_types=[
        [pltpu.VMEM((CHUNK,), jnp.int32) for _ in range(NBUF)],
        [pltpu.VMEM((CHUNK,), jnp.int32) for _ in range(NBUF)],
        [pltpu.VMEM((CHUNK, D_FEAT), jnp.float32) for _ in range(NBUF)],
        pltpu.VMEM((ZROWS, D_FEAT), jnp.float32),
        pltpu.VMEM_SHARED((N_NODES, D_FEAT), jnp.float32),
        [pltpu.SemaphoreType.DMA for _ in range(NBUF)],
        [pltpu.SemaphoreType.DMA for _ in range(NBUF)],
        [pltpu.SemaphoreType.DMA for _ in range(NBUF)],
        pltpu.SemaphoreType.DMA((2,)),
    ],
)


def _sc_deg_body(ei_hbm, deg_hbm, dsts, ones_v, zbuf, deg_sh,
                 isems, ssems, zsems):
    c = lax.axis_index("c")
    s = lax.axis_index("s")
    wid = s * NC + c

    _fill(zbuf, ZROWS, jnp.zeros((L,), jnp.float32))
    _fill(ones_v, CHUNK, jnp.ones((L,), jnp.float32))
    _zero_acc(s, zbuf, deg_sh, zsems)
    plsc.subcore_barrier()

    base = wid * EPW

    def fire_i(k, j):
        off = base + k * CHUNK
        pltpu.async_copy(ei_hbm.at[pl.ds(E_EDGES + off, CHUNK)], dsts[j],
                         isems[j])

    def wait_i(k, j):
        off = base + k * CHUNK
        pltpu.make_async_copy(ei_hbm.at[pl.ds(E_EDGES + off, CHUNK)], dsts[j],
                              isems[j]).wait()

    def fire_s(j):
        pltpu.async_copy(ones_v, deg_sh.at[dsts[j]], ssems[j], add=True)

    def wait_s(j):
        pltpu.make_async_copy(ones_v, deg_sh.at[dsts[j]], ssems[j]).wait()

    def ring(i, _):
        for j in range(NBUF):
            k = NBUF * i + j

            @pl.when(i > 0)
            def _():
                wait_s(j)

            fire_i(k, j)
        for j in range(NBUF):
            wait_i(NBUF * i + j, j)
            fire_s(j)
        return 0

    lax.fori_loop(0, NRING // NBUF, ring, 0)
    for j in range(NBUF):
        wait_s(j)
    for k in range(NRING, NCHUNK):
        j = k - NRING
        fire_i(k, j)
        wait_i(k, j)
        fire_s(j)
        wait_s(j)

    plsc.subcore_barrier()
    _write_out(c, s, deg_sh, deg_hbm, zsems)


_sc_deg_raw = pl.kernel(
    _sc_deg_body,
    out_type=jax.ShapeDtypeStruct((NC, N_NODES, D_FEAT), jnp.float32),
    mesh=_MESH,
    scratch_types=[
        [pltpu.VMEM((CHUNK,), jnp.int32) for _ in range(NBUF)],
        pltpu.VMEM((CHUNK, D_FEAT), jnp.float32),
        pltpu.VMEM((ZROWS, D_FEAT), jnp.float32),
        pltpu.VMEM_SHARED((N_NODES, D_FEAT), jnp.float32),
        [pltpu.SemaphoreType.DMA for _ in range(NBUF)],
        [pltpu.SemaphoreType.DMA for _ in range(NBUF)],
        pltpu.SemaphoreType.DMA((2,)),
    ],
)

ROWS_TC = 1000


def _tc_layer(h, agg, deg, Wr, Wn, b, act):
    def body(h_ref, a0_ref, a1_ref, d0_ref, d1_ref, wr_ref, wn_ref, b_ref,
             o_ref):
        dsum = d0_ref[0, :, :1] + d1_ref[0, :, :1]
        mean = (a0_ref[0] + a1_ref[0]) / jnp.maximum(dsum, 1.0)
        acc = jnp.dot(h_ref[...], wr_ref[...],
                      preferred_element_type=jnp.float32)
        acc = acc + jnp.dot(mean, wn_ref[...],
                            preferred_element_type=jnp.float32)
        acc = acc + b_ref[...]
        if act:
            acc = jnp.maximum(acc, 0.0)
        o_ref[...] = acc

    return pl.pallas_call(
        body,
        grid=(N_NODES // ROWS_TC,),
        in_specs=[
            pl.BlockSpec((ROWS_TC, D_FEAT), lambda i: (i, 0)),
            pl.BlockSpec((1, ROWS_TC, D_FEAT), lambda i: (0, i, 0)),
            pl.BlockSpec((1, ROWS_TC, D_FEAT), lambda i: (1, i, 0)),
            pl.BlockSpec((1, ROWS_TC, D_FEAT), lambda i: (0, i, 0)),
            pl.BlockSpec((1, ROWS_TC, D_FEAT), lambda i: (1, i, 0)),
            pl.BlockSpec((D_FEAT, D_FEAT), lambda i: (0, 0)),
            pl.BlockSpec((D_FEAT, D_FEAT), lambda i: (0, 0)),
            pl.BlockSpec((1, D_FEAT), lambda i: (0, 0)),
        ],
        out_specs=pl.BlockSpec((ROWS_TC, D_FEAT), lambda i: (i, 0)),
        out_shape=jax.ShapeDtypeStruct((N_NODES, D_FEAT), jnp.float32),
    )(h, agg, agg, deg, deg, Wr, Wn, b)


def kernel(x, edge_index, Wr0, Wn0, b0, Wr1, Wn1, b1, Wr2, Wn2, b2):
    ei = edge_index.reshape(2 * E_EDGES)
    deg = _sc_deg_raw(ei)
    agg = _sc_agg_raw(x, ei)
    h = _tc_layer(x, agg, deg, Wr0, Wn0, b0.reshape(1, D_FEAT), True)
    agg = _sc_agg_raw(h, ei)
    h = _tc_layer(h, agg, deg, Wr1, Wn1, b1.reshape(1, D_FEAT), True)
    agg = _sc_agg_raw(h, ei)
    h = _tc_layer(h, agg, deg, Wr2, Wn2, b2.reshape(1, D_FEAT), False)
    return h


# speedup vs baseline: 10.0577x; 1.0013x over previous
import functools

import jax
import jax.numpy as jnp
from jax import lax
from jax.experimental import pallas as pl
from jax.experimental.pallas import tpu as pltpu
from jax.experimental.pallas import tpu_sc as plsc

N_NODES = 10000
D_FEAT = 128
E_EDGES = 320000

NC, NS, L = 2, 16, 16
NW = NC * NS
EPW = E_EDGES // NW
CHUNK = 80
NCHUNK = EPW // CHUNK
NPAIR = (NCHUNK - 1) // 2
RPT = 624
TAIL = N_NODES - NS * RPT
ZROWS = 48

_MESH = plsc.VectorSubcoreMesh(core_axis_name="c", subcore_axis_name="s")


def _fill(buf, rows, vec):
    def row(i, _):
        for j in range(D_FEAT // L):
            buf[i, pl.ds(j * L, L)] = vec
        return 0

    lax.fori_loop(0, rows, row, 0)


def _zero_acc(s, zbuf, acc_sh, zsems):
    def zcopy(t, _):
        @pl.when(t >= 2)
        def _():
            off0 = s * RPT + (t - 2) * ZROWS
            pltpu.make_async_copy(zbuf, acc_sh.at[pl.ds(off0, ZROWS)],
                                  zsems.at[lax.rem(t, 2)]).wait()

        off = s * RPT + t * ZROWS
        pltpu.async_copy(zbuf, acc_sh.at[pl.ds(off, ZROWS)],
                         zsems.at[lax.rem(t, 2)])
        return 0

    nz = RPT // ZROWS
    lax.fori_loop(0, nz, zcopy, 0)
    for t in (nz - 2, nz - 1):
        pltpu.make_async_copy(zbuf, acc_sh.at[pl.ds(s * RPT + t * ZROWS,
                                                    ZROWS)],
                              zsems.at[t % 2]).wait()

    @pl.when(s == 0)
    def _():
        pltpu.sync_copy(zbuf.at[pl.ds(0, TAIL)],
                        acc_sh.at[pl.ds(NS * RPT, TAIL)])


def _write_out(c, s, acc_sh, out_hbm, zsems):
    pltpu.async_copy(acc_sh.at[pl.ds(s * RPT, RPT)],
                     out_hbm.at[c, pl.ds(s * RPT, RPT)], zsems.at[0])

    @pl.when(s == 0)
    def _():
        pltpu.async_copy(acc_sh.at[pl.ds(NS * RPT, TAIL)],
                         out_hbm.at[c, pl.ds(NS * RPT, TAIL)], zsems.at[1])

    pltpu.make_async_copy(acc_sh.at[pl.ds(s * RPT, RPT)],
                          out_hbm.at[c, pl.ds(s * RPT, RPT)],
                          zsems.at[0]).wait()

    @pl.when(s == 0)
    def _():
        pltpu.make_async_copy(acc_sh.at[pl.ds(NS * RPT, TAIL)],
                              out_hbm.at[c, pl.ds(NS * RPT, TAIL)],
                              zsems.at[1]).wait()


NBUF = 4
NRING = (NCHUNK // NBUF) * NBUF


def _sc_agg_body(h_hbm, ei_hbm, agg_hbm,
                 srcs, dsts, rows, zbuf, agg_sh,
                 isems, gsems, ssems, zsems):
    c = lax.axis_index("c")
    s = lax.axis_index("s")
    wid = s * NC + c

    _fill(zbuf, ZROWS, jnp.zeros((L,), jnp.float32))
    _zero_acc(s, zbuf, agg_sh, zsems)
    plsc.subcore_barrier()

    base = wid * EPW

    def fire_i(k, j):
        off = base + k * CHUNK
        pltpu.async_copy(ei_hbm.at[pl.ds(off, CHUNK)], srcs[j], isems[j])
        pltpu.async_copy(ei_hbm.at[pl.ds(E_EDGES + off, CHUNK)], dsts[j],
                         isems[j])

    def wait_i(k, j):
        off = base + k * CHUNK
        pltpu.make_async_copy(ei_hbm.at[pl.ds(off, CHUNK)], srcs[j],
                              isems[j]).wait()
        pltpu.make_async_copy(ei_hbm.at[pl.ds(E_EDGES + off, CHUNK)], dsts[j],
                              isems[j]).wait()

    def fire_g(j):
        pltpu.async_copy(h_hbm.at[srcs[j]], rows[j], gsems[j])

    def wait_g(j):
        pltpu.make_async_copy(h_hbm.at[srcs[j]], rows[j], gsems[j]).wait()

    def fire_s(j):
        pltpu.async_copy(rows[j], agg_sh.at[dsts[j]], ssems[j], add=True)

    def wait_s(j):
        pltpu.make_async_copy(rows[j], agg_sh.at[dsts[j]], ssems[j]).wait()

    def ring(i, _):
        for j in range(NBUF):
            k = NBUF * i + j

            @pl.when(i > 0)
            def _():
                wait_s(j)

            fire_i(k, j)
        for j in range(NBUF):
            wait_i(NBUF * i + j, j)
            fire_g(j)
        for j in range(NBUF):
            wait_g(j)
            fire_s(j)
        return 0

    lax.fori_loop(0, NRING // NBUF, ring, 0)
    for j in range(NBUF):
        wait_s(j)
    for k in range(NRING, NCHUNK):
        j = k - NRING
        fire_i(k, j)
        wait_i(k, j)
        fire_g(j)
        wait_g(j)
        fire_s(j)
        wait_s(j)

    plsc.subcore_barrier()
    _write_out(c, s, agg_sh, agg_hbm, zsems)


_sc_agg_raw = pl.kernel(
    _sc_agg_body,
    out_type=jax.ShapeDtypeStruct((NC, N_NODES, D_FEAT), jnp.float32),
    mesh=_MESH,
    scratch_types=[
        [pltpu.VMEM((CHUNK,), jnp.int32) for _ in range(NBUF)],
        [pltpu.VMEM((CHUNK,), jnp.int32) for _ in range(NBUF)],
        [pltpu.VMEM((CHUNK, D_FEAT), jnp.float32) for _ in range(NBUF)],
        pltpu.VMEM((ZROWS, D_FEAT), jnp.float32),
        pltpu.VMEM_SHARED((N_NODES, D_FEAT), jnp.float32),
        [pltpu.SemaphoreType.DMA for _ in range(NBUF)],
        [pltpu.SemaphoreType.DMA for _ in range(NBUF)],
        [pltpu.SemaphoreType.DMA for _ in range(NBUF)],
        pltpu.SemaphoreType.DMA((2,)),
    ],
)


def _sc_deg_body(ei_hbm, deg_hbm, dsts, ones_v, zbuf, deg_sh,
                 isems, ssems, zsems):
    c = lax.axis_index("c")
    s = lax.axis_index("s")
    wid = s * NC + c

    _fill(zbuf, ZROWS, jnp.zeros((L,), jnp.float32))
    _fill(ones_v, CHUNK, jnp.ones((L,), jnp.float32))
    _zero_acc(s, zbuf, deg_sh, zsems)
    plsc.subcore_barrier()

    base = wid * EPW

    def fire_i(k, j):
        off = base + k * CHUNK
        pltpu.async_copy(ei_hbm.at[pl.ds(E_EDGES + off, CHUNK)], dsts[j],
                         isems[j])

    def wait_i(k, j):
        off = base + k * CHUNK
        pltpu.make_async_copy(ei_hbm.at[pl.ds(E_EDGES + off, CHUNK)], dsts[j],
                              isems[j]).wait()

    def fire_s(j):
        pltpu.async_copy(ones_v, deg_sh.at[dsts[j]], ssems[j], add=True)

    def wait_s(j):
        pltpu.make_async_copy(ones_v, deg_sh.at[dsts[j]], ssems[j]).wait()

    def ring(i, _):
        for j in range(NBUF):
            k = NBUF * i + j

            @pl.when(i > 0)
            def _():
                wait_s(j)

            fire_i(k, j)
        for j in range(NBUF):
            wait_i(NBUF * i + j, j)
            fire_s(j)
        return 0

    lax.fori_loop(0, NRING // NBUF, ring, 0)
    for j in range(NBUF):
        wait_s(j)
    for k in range(NRING, NCHUNK):
        j = k - NRING
        fire_i(k, j)
        wait_i(k, j)
        fire_s(j)
        wait_s(j)

    plsc.subcore_barrier()
    _write_out(c, s, deg_sh, deg_hbm, zsems)


_sc_deg_raw = pl.kernel(
    _sc_deg_body,
    out_type=jax.ShapeDtypeStruct((NC, N_NODES, D_FEAT), jnp.float32),
    mesh=_MESH,
    scratch_types=[
        [pltpu.VMEM((CHUNK,), jnp.int32) for _ in range(NBUF)],
        pltpu.VMEM((CHUNK, D_FEAT), jnp.float32),
        pltpu.VMEM((ZROWS, D_FEAT), jnp.float32),
        pltpu.VMEM_SHARED((N_NODES, D_FEAT), jnp.float32),
        [pltpu.SemaphoreType.DMA for _ in range(NBUF)],
        [pltpu.SemaphoreType.DMA for _ in range(NBUF)],
        pltpu.SemaphoreType.DMA((2,)),
    ],
)

ROWS_TC = 1000


def _tc_root(h, Wr, b):
    def body(h_ref, wr_ref, b_ref, o_ref):
        o_ref[...] = jnp.dot(h_ref[...], wr_ref[...],
                             preferred_element_type=jnp.float32) + b_ref[...]

    return pl.pallas_call(
        body,
        grid=(N_NODES // ROWS_TC,),
        in_specs=[
            pl.BlockSpec((ROWS_TC, D_FEAT), lambda i: (i, 0)),
            pl.BlockSpec((D_FEAT, D_FEAT), lambda i: (0, 0)),
            pl.BlockSpec((1, D_FEAT), lambda i: (0, 0)),
        ],
        out_specs=pl.BlockSpec((ROWS_TC, D_FEAT), lambda i: (i, 0)),
        out_shape=jax.ShapeDtypeStruct((N_NODES, D_FEAT), jnp.float32),
    )(h, Wr, b)


def _tc_combine(hr, agg, deg, Wn, act):
    def body(hr_ref, a0_ref, a1_ref, d0_ref, d1_ref, wn_ref, o_ref):
        dsum = d0_ref[0, :, :1] + d1_ref[0, :, :1]
        mean = (a0_ref[0] + a1_ref[0]) / jnp.maximum(dsum, 1.0)
        acc = hr_ref[...] + jnp.dot(mean, wn_ref[...],
                                    preferred_element_type=jnp.float32)
        if act:
            acc = jnp.maximum(acc, 0.0)
        o_ref[...] = acc

    return pl.pallas_call(
        body,
        grid=(N_NODES // ROWS_TC,),
        in_specs=[
            pl.BlockSpec((ROWS_TC, D_FEAT), lambda i: (i, 0)),
            pl.BlockSpec((1, ROWS_TC, D_FEAT), lambda i: (0, i, 0)),
            pl.BlockSpec((1, ROWS_TC, D_FEAT), lambda i: (1, i, 0)),
            pl.BlockSpec((1, ROWS_TC, D_FEAT), lambda i: (0, i, 0)),
            pl.BlockSpec((1, ROWS_TC, D_FEAT), lambda i: (1, i, 0)),
            pl.BlockSpec((D_FEAT, D_FEAT), lambda i: (0, 0)),
        ],
        out_specs=pl.BlockSpec((ROWS_TC, D_FEAT), lambda i: (i, 0)),
        out_shape=jax.ShapeDtypeStruct((N_NODES, D_FEAT), jnp.float32),
    )(hr, agg, agg, deg, deg, Wn)


def kernel(x, edge_index, Wr0, Wn0, b0, Wr1, Wn1, b1, Wr2, Wn2, b2):
    ei = edge_index.reshape(2 * E_EDGES)
    deg = _sc_deg_raw(ei)
    agg = _sc_agg_raw(x, ei)
    hr = _tc_root(x, Wr0, b0.reshape(1, D_FEAT))
    h = _tc_combine(hr, agg, deg, Wn0, True)
    agg = _sc_agg_raw(h, ei)
    hr = _tc_root(h, Wr1, b1.reshape(1, D_FEAT))
    h = _tc_combine(hr, agg, deg, Wn1, True)
    agg = _sc_agg_raw(h, ei)
    hr = _tc_root(h, Wr2, b2.reshape(1, D_FEAT))
    h = _tc_combine(hr, agg, deg, Wn2, False)
    return h


# speedup vs baseline: 10.1566x; 1.0098x over previous
import functools

import jax
import jax.numpy as jnp
from jax import lax
from jax.experimental import pallas as pl
from jax.experimental.pallas import tpu as pltpu
from jax.experimental.pallas import tpu_sc as plsc

N_NODES = 10000
D_FEAT = 128
E_EDGES = 320000

NC, NS, L = 2, 16, 16
NW = NC * NS
EPW = E_EDGES // NW
CHUNK = 80
NCHUNK = EPW // CHUNK
NPAIR = (NCHUNK - 1) // 2
RPT = 624
TAIL = N_NODES - NS * RPT
ZROWS = 48

_MESH = plsc.VectorSubcoreMesh(core_axis_name="c", subcore_axis_name="s")


def _fill(buf, rows, vec):
    def row(i, _):
        for j in range(D_FEAT // L):
            buf[i, pl.ds(j * L, L)] = vec
        return 0

    lax.fori_loop(0, rows, row, 0)


def _zero_acc(s, zbuf, acc_sh, zsems):
    def zcopy(t, _):
        @pl.when(t >= 2)
        def _():
            off0 = s * RPT + (t - 2) * ZROWS
            pltpu.make_async_copy(zbuf, acc_sh.at[pl.ds(off0, ZROWS)],
                                  zsems.at[lax.rem(t, 2)]).wait()

        off = s * RPT + t * ZROWS
        pltpu.async_copy(zbuf, acc_sh.at[pl.ds(off, ZROWS)],
                         zsems.at[lax.rem(t, 2)])
        return 0

    nz = RPT // ZROWS
    lax.fori_loop(0, nz, zcopy, 0)
    for t in (nz - 2, nz - 1):
        pltpu.make_async_copy(zbuf, acc_sh.at[pl.ds(s * RPT + t * ZROWS,
                                                    ZROWS)],
                              zsems.at[t % 2]).wait()

    @pl.when(s == 0)
    def _():
        pltpu.sync_copy(zbuf.at[pl.ds(0, TAIL)],
                        acc_sh.at[pl.ds(NS * RPT, TAIL)])


def _write_out(c, s, acc_sh, out_hbm, zsems):
    pltpu.async_copy(acc_sh.at[pl.ds(s * RPT, RPT)],
                     out_hbm.at[c, pl.ds(s * RPT, RPT)], zsems.at[0])

    @pl.when(s == 0)
    def _():
        pltpu.async_copy(acc_sh.at[pl.ds(NS * RPT, TAIL)],
                         out_hbm.at[c, pl.ds(NS * RPT, TAIL)], zsems.at[1])

    pltpu.make_async_copy(acc_sh.at[pl.ds(s * RPT, RPT)],
                          out_hbm.at[c, pl.ds(s * RPT, RPT)],
                          zsems.at[0]).wait()

    @pl.when(s == 0)
    def _():
        pltpu.make_async_copy(acc_sh.at[pl.ds(NS * RPT, TAIL)],
                              out_hbm.at[c, pl.ds(NS * RPT, TAIL)],
                              zsems.at[1]).wait()


NBUF = 4
NRING = (NCHUNK // NBUF) * NBUF


def _sc_agg_body(h_hbm, ei_hbm, agg_hbm,
                 eis, rows, zbuf, agg_sh,
                 isems, gsems, ssems, zsems):
    c = lax.axis_index("c")
    s = lax.axis_index("s")
    wid = s * NC + c

    _fill(zbuf, ZROWS, jnp.zeros((L,), jnp.float32))
    _zero_acc(s, zbuf, agg_sh, zsems)
    plsc.subcore_barrier()

    base = wid * EPW

    def fire_i(k, j):
        off = base + k * CHUNK
        pltpu.async_copy(ei_hbm.at[pl.ds(off, CHUNK)], eis[j].at[0],
                         isems[j])
        pltpu.async_copy(ei_hbm.at[pl.ds(E_EDGES + off, CHUNK)],
                         eis[j].at[1], isems[j])

    def wait_i(k, j):
        off = base + k * CHUNK
        pltpu.make_async_copy(ei_hbm.at[pl.ds(off, CHUNK)], eis[j].at[0],
                              isems[j]).wait()
        pltpu.make_async_copy(ei_hbm.at[pl.ds(E_EDGES + off, CHUNK)],
                              eis[j].at[1], isems[j]).wait()

    def fire_g(j):
        pltpu.async_copy(h_hbm.at[eis[j].at[0]], rows[j], gsems[j])

    def wait_g(j):
        pltpu.make_async_copy(h_hbm.at[eis[j].at[0]], rows[j],
                              gsems[j]).wait()

    def fire_s(j):
        pltpu.async_copy(rows[j], agg_sh.at[eis[j].at[1]], ssems[j],
                         add=True)

    def wait_s(j):
        pltpu.make_async_copy(rows[j], agg_sh.at[eis[j].at[1]],
                              ssems[j]).wait()

    def ring(i, _):
        for j in range(NBUF):
            k = NBUF * i + j

            @pl.when(i > 0)
            def _():
                wait_s(j)

            fire_i(k, j)
        for j in range(NBUF):
            wait_i(NBUF * i + j, j)
            fire_g(j)
        for j in range(NBUF):
            wait_g(j)
            fire_s(j)
        return 0

    lax.fori_loop(0, NRING // NBUF, ring, 0)
    for j in range(NBUF):
        wait_s(j)
    for k in range(NRING, NCHUNK):
        j = k - NRING
        fire_i(k, j)
        wait_i(k, j)
        fire_g(j)
        wait_g(j)
        fire_s(j)
        wait_s(j)

    plsc.subcore_barrier()
    _write_out(c, s, agg_sh, agg_hbm, zsems)


_sc_agg_raw = pl.kernel(
    _sc_agg_body,
    out_type=jax.ShapeDtypeStruct((NC, N_NODES, D_FEAT), jnp.float32),
    mesh=_MESH,
    scratch_types=[
        [pltpu.VMEM((2, CHUNK), jnp.int32) for _ in range(NBUF)],
        [pltpu.VMEM((CHUNK, D_FEAT), jnp.float32) for _ in range(NBUF)],
        pltpu.VMEM((ZROWS, D_FEAT), jnp.float32),
        pltpu.VMEM_SHARED((N_NODES, D_FEAT), jnp.float32),
        [pltpu.SemaphoreType.DMA for _ in range(NBUF)],
        [pltpu.SemaphoreType.DMA for _ in range(NBUF)],
        [pltpu.SemaphoreType.DMA for _ in range(NBUF)],
        pltpu.SemaphoreType.DMA((2,)),
    ],
)


def _sc_deg_body(ei_hbm, deg_hbm, eis, ones_v, zbuf, deg_sh,
                 isems, ssems, zsems):
    c = lax.axis_index("c")
    s = lax.axis_index("s")
    wid = s * NC + c

    _fill(zbuf, ZROWS, jnp.zeros((L,), jnp.float32))
    _fill(ones_v, CHUNK, jnp.ones((L,), jnp.float32))
    _zero_acc(s, zbuf, deg_sh, zsems)
    plsc.subcore_barrier()

    base = wid * EPW

    def fire_i(k, j):
        off = base + k * CHUNK
        pltpu.async_copy(ei_hbm.at[pl.ds(E_EDGES + off, CHUNK)],
                         eis[j].at[1], isems[j])

    def wait_i(k, j):
        off = base + k * CHUNK
        pltpu.make_async_copy(ei_hbm.at[pl.ds(E_EDGES + off, CHUNK)],
                              eis[j].at[1], isems[j]).wait()

    def fire_s(j):
        pltpu.async_copy(ones_v, deg_sh.at[eis[j].at[1]], ssems[j], add=True)

    def wait_s(j):
        pltpu.make_async_copy(ones_v, deg_sh.at[eis[j].at[1]],
                              ssems[j]).wait()

    def ring(i, _):
        for j in range(NBUF):
            k = NBUF * i + j

            @pl.when(i > 0)
            def _():
                wait_s(j)

            fire_i(k, j)
        for j in range(NBUF):
            wait_i(NBUF * i + j, j)
            fire_s(j)
        return 0

    lax.fori_loop(0, NRING // NBUF, ring, 0)
    for j in range(NBUF):
        wait_s(j)
    for k in range(NRING, NCHUNK):
        j = k - NRING
        fire_i(k, j)
        wait_i(k, j)
        fire_s(j)
        wait_s(j)

    plsc.subcore_barrier()
    _write_out(c, s, deg_sh, deg_hbm, zsems)


_sc_deg_raw = pl.kernel(
    _sc_deg_body,
    out_type=jax.ShapeDtypeStruct((NC, N_NODES, D_FEAT), jnp.float32),
    mesh=_MESH,
    scratch_types=[
        [pltpu.VMEM((2, CHUNK), jnp.int32) for _ in range(NBUF)],
        pltpu.VMEM((CHUNK, D_FEAT), jnp.float32),
        pltpu.VMEM((ZROWS, D_FEAT), jnp.float32),
        pltpu.VMEM_SHARED((N_NODES, D_FEAT), jnp.float32),
        [pltpu.SemaphoreType.DMA for _ in range(NBUF)],
        [pltpu.SemaphoreType.DMA for _ in range(NBUF)],
        pltpu.SemaphoreType.DMA((2,)),
    ],
)

ROWS_TC = 2000


def _tc_root(h, Wr, b):
    def body(h_ref, wr_ref, b_ref, o_ref):
        o_ref[...] = jnp.dot(h_ref[...], wr_ref[...],
                             preferred_element_type=jnp.float32) + b_ref[...]

    return pl.pallas_call(
        body,
        grid=(N_NODES // ROWS_TC,),
        in_specs=[
            pl.BlockSpec((ROWS_TC, D_FEAT), lambda i: (i, 0)),
            pl.BlockSpec((D_FEAT, D_FEAT), lambda i: (0, 0)),
            pl.BlockSpec((1, D_FEAT), lambda i: (0, 0)),
        ],
        out_specs=pl.BlockSpec((ROWS_TC, D_FEAT), lambda i: (i, 0)),
        out_shape=jax.ShapeDtypeStruct((N_NODES, D_FEAT), jnp.float32),
    )(h, Wr, b)


def _tc_combine(hr, agg, deg, Wn, act):
    def body(hr_ref, a0_ref, a1_ref, d0_ref, d1_ref, wn_ref, o_ref):
        dsum = d0_ref[0, :, :1] + d1_ref[0, :, :1]
        mean = (a0_ref[0] + a1_ref[0]) / jnp.maximum(dsum, 1.0)
        acc = hr_ref[...] + jnp.dot(mean, wn_ref[...],
                                    preferred_element_type=jnp.float32)
        if act:
            acc = jnp.maximum(acc, 0.0)
        o_ref[...] = acc

    return pl.pallas_call(
        body,
        grid=(N_NODES // ROWS_TC,),
        in_specs=[
            pl.BlockSpec((ROWS_TC, D_FEAT), lambda i: (i, 0)),
            pl.BlockSpec((1, ROWS_TC, D_FEAT), lambda i: (0, i, 0)),
            pl.BlockSpec((1, ROWS_TC, D_FEAT), lambda i: (1, i, 0)),
            pl.BlockSpec((1, ROWS_TC, D_FEAT), lambda i: (0, i, 0)),
            pl.BlockSpec((1, ROWS_TC, D_FEAT), lambda i: (1, i, 0)),
            pl.BlockSpec((D_FEAT, D_FEAT), lambda i: (0, 0)),
        ],
        out_specs=pl.BlockSpec((ROWS_TC, D_FEAT), lambda i: (i, 0)),
        out_shape=jax.ShapeDtypeStruct((N_NODES, D_FEAT), jnp.float32),
    )(hr, agg, agg, deg, deg, Wn)


def kernel(x, edge_index, Wr0, Wn0, b0, Wr1, Wn1, b1, Wr2, Wn2, b2):
    ei = edge_index.reshape(2 * E_EDGES)
    deg = _sc_deg_raw(ei)
    agg = _sc_agg_raw(x, ei)
    hr = _tc_root(x, Wr0, b0.reshape(1, D_FEAT))
    h = _tc_combine(hr, agg, deg, Wn0, True)
    agg = _sc_agg_raw(h, ei)
    hr = _tc_root(h, Wr1, b1.reshape(1, D_FEAT))
    h = _tc_combine(hr, agg, deg, Wn1, True)
    agg = _sc_agg_raw(h, ei)
    hr = _tc_root(h, Wr2, b2.reshape(1, D_FEAT))
    h = _tc_combine(hr, agg, deg, Wn2, False)
    return h


# speedup vs baseline: 10.2679x; 1.0110x over previous
import functools

import jax
import jax.numpy as jnp
from jax import lax
from jax.experimental import pallas as pl
from jax.experimental.pallas import tpu as pltpu
from jax.experimental.pallas import tpu_sc as plsc

N_NODES = 10000
D_FEAT = 128
E_EDGES = 320000

NC, NS, L = 2, 16, 16
NW = NC * NS
EPW = E_EDGES // NW
CHUNK = 80
NCHUNK = EPW // CHUNK
NPAIR = (NCHUNK - 1) // 2
RPT = 624
TAIL = N_NODES - NS * RPT
ZROWS = 48

_MESH = plsc.VectorSubcoreMesh(core_axis_name="c", subcore_axis_name="s")


def _fill(buf, rows, vec):
    def row(i, _):
        for j in range(D_FEAT // L):
            buf[i, pl.ds(j * L, L)] = vec
        return 0

    lax.fori_loop(0, rows, row, 0)


def _zero_acc(s, zbuf, acc_sh, zsems):
    def zcopy(t, _):
        @pl.when(t >= 2)
        def _():
            off0 = s * RPT + (t - 2) * ZROWS
            pltpu.make_async_copy(zbuf, acc_sh.at[pl.ds(off0, ZROWS)],
                                  zsems.at[lax.rem(t, 2)]).wait()

        off = s * RPT + t * ZROWS
        pltpu.async_copy(zbuf, acc_sh.at[pl.ds(off, ZROWS)],
                         zsems.at[lax.rem(t, 2)])
        return 0

    nz = RPT // ZROWS
    lax.fori_loop(0, nz, zcopy, 0)
    for t in (nz - 2, nz - 1):
        pltpu.make_async_copy(zbuf, acc_sh.at[pl.ds(s * RPT + t * ZROWS,
                                                    ZROWS)],
                              zsems.at[t % 2]).wait()

    @pl.when(s == 0)
    def _():
        pltpu.sync_copy(zbuf.at[pl.ds(0, TAIL)],
                        acc_sh.at[pl.ds(NS * RPT, TAIL)])


def _write_out(c, s, acc_sh, out_hbm, zsems):
    pltpu.async_copy(acc_sh.at[pl.ds(s * RPT, RPT)],
                     out_hbm.at[c, pl.ds(s * RPT, RPT)], zsems.at[0])

    @pl.when(s == 0)
    def _():
        pltpu.async_copy(acc_sh.at[pl.ds(NS * RPT, TAIL)],
                         out_hbm.at[c, pl.ds(NS * RPT, TAIL)], zsems.at[1])

    pltpu.make_async_copy(acc_sh.at[pl.ds(s * RPT, RPT)],
                          out_hbm.at[c, pl.ds(s * RPT, RPT)],
                          zsems.at[0]).wait()

    @pl.when(s == 0)
    def _():
        pltpu.make_async_copy(acc_sh.at[pl.ds(NS * RPT, TAIL)],
                              out_hbm.at[c, pl.ds(NS * RPT, TAIL)],
                              zsems.at[1]).wait()


NBUF = 4
NRING = (NCHUNK // NBUF) * NBUF


def _sc_agg_body(h_hbm, ei_hbm, agg_hbm,
                 eis, rows, zbuf, agg_sh,
                 isems, gsems, ssems, zsems):
    c = lax.axis_index("c")
    s = lax.axis_index("s")
    wid = s * NC + c

    _fill(zbuf, ZROWS, jnp.zeros((L,), jnp.float32))
    _zero_acc(s, zbuf, agg_sh, zsems)
    plsc.subcore_barrier()

    base = wid * EPW

    def fire_i(k, j):
        off = base + k * CHUNK
        pltpu.async_copy(ei_hbm.at[pl.ds(off, CHUNK)], eis[j].at[0],
                         isems[j])
        pltpu.async_copy(ei_hbm.at[pl.ds(E_EDGES + off, CHUNK)],
                         eis[j].at[1], isems[j])

    def wait_i(k, j):
        off = base + k * CHUNK
        pltpu.make_async_copy(ei_hbm.at[pl.ds(off, CHUNK)], eis[j].at[0],
                              isems[j]).wait()
        pltpu.make_async_copy(ei_hbm.at[pl.ds(E_EDGES + off, CHUNK)],
                              eis[j].at[1], isems[j]).wait()

    def fire_g(j):
        pltpu.async_copy(h_hbm.at[eis[j].at[0]], rows[j], gsems[j])

    def wait_g(j):
        pltpu.make_async_copy(h_hbm.at[eis[j].at[0]], rows[j],
                              gsems[j]).wait()

    def fire_s(j):
        pltpu.async_copy(rows[j], agg_sh.at[eis[j].at[1]], ssems[j],
                         add=True)

    def wait_s(j):
        pltpu.make_async_copy(rows[j], agg_sh.at[eis[j].at[1]],
                              ssems[j]).wait()

    def ring(i, _):
        for j in range(NBUF):
            k = NBUF * i + j

            @pl.when(i > 0)
            def _():
                wait_s(j)

            fire_i(k, j)
        for j in range(NBUF):
            wait_i(NBUF * i + j, j)
            fire_g(j)
        for j in range(NBUF):
            wait_g(j)
            fire_s(j)
        return 0

    lax.fori_loop(0, NRING // NBUF, ring, 0)
    for j in range(NBUF):
        wait_s(j)
    for k in range(NRING, NCHUNK):
        j = k - NRING
        fire_i(k, j)
        wait_i(k, j)
        fire_g(j)
        wait_g(j)
        fire_s(j)
        wait_s(j)

    plsc.subcore_barrier()
    _write_out(c, s, agg_sh, agg_hbm, zsems)


_sc_agg_raw = pl.kernel(
    _sc_agg_body,
    out_type=jax.ShapeDtypeStruct((NC, N_NODES, D_FEAT), jnp.float32),
    mesh=_MESH,
    scratch_types=[
        [pltpu.VMEM((2, CHUNK), jnp.int32) for _ in range(NBUF)],
        [pltpu.VMEM((CHUNK, D_FEAT), jnp.float32) for _ in range(NBUF)],
        pltpu.VMEM((ZROWS, D_FEAT), jnp.float32),
        pltpu.VMEM_SHARED((N_NODES, D_FEAT), jnp.float32),
        [pltpu.SemaphoreType.DMA for _ in range(NBUF)],
        [pltpu.SemaphoreType.DMA for _ in range(NBUF)],
        [pltpu.SemaphoreType.DMA for _ in range(NBUF)],
        pltpu.SemaphoreType.DMA((2,)),
    ],
)


def _sc_deg_body(ei_hbm, deg_hbm, eis, ones_v, zbuf, deg_sh,
                 isems, ssems, zsems):
    c = lax.axis_index("c")
    s = lax.axis_index("s")
    wid = s * NC + c

    _fill(zbuf, ZROWS, jnp.zeros((L,), jnp.float32))
    _fill(ones_v, CHUNK, jnp.ones((L,), jnp.float32))
    _zero_acc(s, zbuf, deg_sh, zsems)
    plsc.subcore_barrier()

    base = wid * EPW

    def fire_i(k, j):
        off = base + k * CHUNK
        pltpu.async_copy(ei_hbm.at[pl.ds(E_EDGES + off, CHUNK)],
                         eis[j].at[1], isems[j])

    def wait_i(k, j):
        off = base + k * CHUNK
        pltpu.make_async_copy(ei_hbm.at[pl.ds(E_EDGES + off, CHUNK)],
                              eis[j].at[1], isems[j]).wait()

    def fire_s(j):
        pltpu.async_copy(ones_v, deg_sh.at[eis[j].at[1]], ssems[j], add=True)

    def wait_s(j):
        pltpu.make_async_copy(ones_v, deg_sh.at[eis[j].at[1]],
                              ssems[j]).wait()

    def ring(i, _):
        for j in range(NBUF):
            k = NBUF * i + j

            @pl.when(i > 0)
            def _():
                wait_s(j)

            fire_i(k, j)
        for j in range(NBUF):
            wait_i(NBUF * i + j, j)
            fire_s(j)
        return 0

    lax.fori_loop(0, NRING // NBUF, ring, 0)
    for j in range(NBUF):
        wait_s(j)
    for k in range(NRING, NCHUNK):
        j = k - NRING
        fire_i(k, j)
        wait_i(k, j)
        fire_s(j)
        wait_s(j)

    plsc.subcore_barrier()
    _write_out(c, s, deg_sh, deg_hbm, zsems)


_sc_deg_raw = pl.kernel(
    _sc_deg_body,
    out_type=jax.ShapeDtypeStruct((NC, N_NODES, D_FEAT), jnp.float32),
    mesh=_MESH,
    scratch_types=[
        [pltpu.VMEM((2, CHUNK), jnp.int32) for _ in range(NBUF)],
        pltpu.VMEM((CHUNK, D_FEAT), jnp.float32),
        pltpu.VMEM((ZROWS, D_FEAT), jnp.float32),
        pltpu.VMEM_SHARED((N_NODES, D_FEAT), jnp.float32),
        [pltpu.SemaphoreType.DMA for _ in range(NBUF)],
        [pltpu.SemaphoreType.DMA for _ in range(NBUF)],
        pltpu.SemaphoreType.DMA((2,)),
    ],
)

ROWS_TC = 2000


def _tc_root(h, Wr, b):
    def body(h_ref, wr_ref, b_ref, o_ref):
        o_ref[...] = jnp.dot(h_ref[...], wr_ref[...],
                             preferred_element_type=jnp.float32) + b_ref[...]

    return pl.pallas_call(
        body,
        grid=(N_NODES // ROWS_TC,),
        in_specs=[
            pl.BlockSpec((ROWS_TC, D_FEAT), lambda i: (i, 0)),
            pl.BlockSpec((D_FEAT, D_FEAT), lambda i: (0, 0)),
            pl.BlockSpec((1, D_FEAT), lambda i: (0, 0)),
        ],
        out_specs=pl.BlockSpec((ROWS_TC, D_FEAT), lambda i: (i, 0)),
        out_shape=jax.ShapeDtypeStruct((N_NODES, D_FEAT), jnp.float32),
    )(h, Wr, b)


def _tc_recip(deg):
    def body(d0_ref, d1_ref, o_ref):
        o_ref[...] = 1.0 / jnp.maximum(d0_ref[0] + d1_ref[0], 1.0)

    return pl.pallas_call(
        body,
        grid=(N_NODES // ROWS_TC,),
        in_specs=[
            pl.BlockSpec((1, ROWS_TC, D_FEAT), lambda i: (0, i, 0)),
            pl.BlockSpec((1, ROWS_TC, D_FEAT), lambda i: (1, i, 0)),
        ],
        out_specs=pl.BlockSpec((ROWS_TC, D_FEAT), lambda i: (i, 0)),
        out_shape=jax.ShapeDtypeStruct((N_NODES, D_FEAT), jnp.float32),
    )(deg, deg)


def _tc_combine(hr, agg, recip, Wn, act):
    def body(hr_ref, a0_ref, a1_ref, r_ref, wn_ref, o_ref):
        mean = (a0_ref[0] + a1_ref[0]) * r_ref[...]
        acc = hr_ref[...] + jnp.dot(mean, wn_ref[...],
                                    preferred_element_type=jnp.float32)
        if act:
            acc = jnp.maximum(acc, 0.0)
        o_ref[...] = acc

    return pl.pallas_call(
        body,
        grid=(N_NODES // ROWS_TC,),
        in_specs=[
            pl.BlockSpec((ROWS_TC, D_FEAT), lambda i: (i, 0)),
            pl.BlockSpec((1, ROWS_TC, D_FEAT), lambda i: (0, i, 0)),
            pl.BlockSpec((1, ROWS_TC, D_FEAT), lambda i: (1, i, 0)),
            pl.BlockSpec((ROWS_TC, D_FEAT), lambda i: (i, 0)),
            pl.BlockSpec((D_FEAT, D_FEAT), lambda i: (0, 0)),
        ],
        out_specs=pl.BlockSpec((ROWS_TC, D_FEAT), lambda i: (i, 0)),
        out_shape=jax.ShapeDtypeStruct((N_NODES, D_FEAT), jnp.float32),
    )(hr, agg, agg, recip, Wn)


def kernel(x, edge_index, Wr0, Wn0, b0, Wr1, Wn1, b1, Wr2, Wn2, b2):
    ei = edge_index.reshape(2 * E_EDGES)
    deg = _sc_deg_raw(ei)
    agg = _sc_agg_raw(x, ei)
    recip = _tc_recip(deg)
    hr = _tc_root(x, Wr0, b0.reshape(1, D_FEAT))
    h = _tc_combine(hr, agg, recip, Wn0, True)
    agg = _sc_agg_raw(h, ei)
    hr = _tc_root(h, Wr1, b1.reshape(1, D_FEAT))
    h = _tc_combine(hr, agg, recip, Wn1, True)
    agg = _sc_agg_raw(h, ei)
    hr = _tc_root(h, Wr2, b2.reshape(1, D_FEAT))
    h = _tc_combine(hr, agg, recip, Wn2, False)
    return h


# speedup vs baseline: 10.3155x; 1.0046x over previous
import functools

import jax
import jax.numpy as jnp
from jax import lax
from jax.experimental import pallas as pl
from jax.experimental.pallas import tpu as pltpu
from jax.experimental.pallas import tpu_sc as plsc

N_NODES = 10000
D_FEAT = 128
E_EDGES = 320000

NC, NS, L = 2, 16, 16
NW = NC * NS
EPW = E_EDGES // NW
CHUNK = 80
NCHUNK = EPW // CHUNK
NPAIR = (NCHUNK - 1) // 2
RPT = 624
TAIL = N_NODES - NS * RPT
ZROWS = 48

_MESH = plsc.VectorSubcoreMesh(core_axis_name="c", subcore_axis_name="s")


def _fill(buf, rows, vec):
    def row(i, _):
        for j in range(D_FEAT // L):
            buf[i, pl.ds(j * L, L)] = vec
        return 0

    lax.fori_loop(0, rows, row, 0)


def _zero_acc(s, zbuf, acc_sh, zsems):
    def zcopy(t, _):
        @pl.when(t >= 2)
        def _():
            off0 = s * RPT + (t - 2) * ZROWS
            pltpu.make_async_copy(zbuf, acc_sh.at[pl.ds(off0, ZROWS)],
                                  zsems.at[lax.rem(t, 2)]).wait()

        off = s * RPT + t * ZROWS
        pltpu.async_copy(zbuf, acc_sh.at[pl.ds(off, ZROWS)],
                         zsems.at[lax.rem(t, 2)])
        return 0

    nz = RPT // ZROWS
    lax.fori_loop(0, nz, zcopy, 0)
    for t in (nz - 2, nz - 1):
        pltpu.make_async_copy(zbuf, acc_sh.at[pl.ds(s * RPT + t * ZROWS,
                                                    ZROWS)],
                              zsems.at[t % 2]).wait()

    @pl.when(s == 0)
    def _():
        pltpu.sync_copy(zbuf.at[pl.ds(0, TAIL)],
                        acc_sh.at[pl.ds(NS * RPT, TAIL)])


def _write_out(c, s, acc_sh, out_hbm, zsems):
    pltpu.async_copy(acc_sh.at[pl.ds(s * RPT, RPT)],
                     out_hbm.at[c, pl.ds(s * RPT, RPT)], zsems.at[0])

    @pl.when(s == 0)
    def _():
        pltpu.async_copy(acc_sh.at[pl.ds(NS * RPT, TAIL)],
                         out_hbm.at[c, pl.ds(NS * RPT, TAIL)], zsems.at[1])

    pltpu.make_async_copy(acc_sh.at[pl.ds(s * RPT, RPT)],
                          out_hbm.at[c, pl.ds(s * RPT, RPT)],
                          zsems.at[0]).wait()

    @pl.when(s == 0)
    def _():
        pltpu.make_async_copy(acc_sh.at[pl.ds(NS * RPT, TAIL)],
                              out_hbm.at[c, pl.ds(NS * RPT, TAIL)],
                              zsems.at[1]).wait()


NBUF = 4
NRING = (NCHUNK // NBUF) * NBUF


def _sc_agg_body(h_hbm, ei_hbm, agg_hbm,
                 eis, rows, zbuf, agg_sh,
                 isems, gsems, ssems, zsems):
    c = lax.axis_index("c")
    s = lax.axis_index("s")
    wid = s * NC + c

    base = wid * EPW

    def fire_i(k, j):
        off = base + k * CHUNK
        pltpu.async_copy(ei_hbm.at[pl.ds(off, CHUNK)], eis[j].at[0],
                         isems[j])
        pltpu.async_copy(ei_hbm.at[pl.ds(E_EDGES + off, CHUNK)],
                         eis[j].at[1], isems[j])

    def wait_i(k, j):
        off = base + k * CHUNK
        pltpu.make_async_copy(ei_hbm.at[pl.ds(off, CHUNK)], eis[j].at[0],
                              isems[j]).wait()
        pltpu.make_async_copy(ei_hbm.at[pl.ds(E_EDGES + off, CHUNK)],
                              eis[j].at[1], isems[j]).wait()

    def fire_g(j):
        pltpu.async_copy(h_hbm.at[eis[j].at[0]], rows[j], gsems[j])

    def wait_g(j):
        pltpu.make_async_copy(h_hbm.at[eis[j].at[0]], rows[j],
                              gsems[j]).wait()

    def fire_s(j):
        pltpu.async_copy(rows[j], agg_sh.at[eis[j].at[1]], ssems[j],
                         add=True)

    def wait_s(j):
        pltpu.make_async_copy(rows[j], agg_sh.at[eis[j].at[1]],
                              ssems[j]).wait()

    for j in range(NBUF):
        fire_i(j, j)
    _fill(zbuf, ZROWS, jnp.zeros((L,), jnp.float32))
    _zero_acc(s, zbuf, agg_sh, zsems)
    plsc.subcore_barrier()

    def ring(i, _):
        for j in range(NBUF):
            k = NBUF * i + j

            @pl.when(i > 0)
            def _():
                wait_s(j)
                fire_i(k, j)

        for j in range(NBUF):
            wait_i(NBUF * i + j, j)
            fire_g(j)
        for j in range(NBUF):
            wait_g(j)
            fire_s(j)
        return 0

    lax.fori_loop(0, NRING // NBUF, ring, 0)
    for j in range(NBUF):
        wait_s(j)
    for k in range(NRING, NCHUNK):
        j = k - NRING
        fire_i(k, j)
        wait_i(k, j)
        fire_g(j)
        wait_g(j)
        fire_s(j)
        wait_s(j)

    plsc.subcore_barrier()
    _write_out(c, s, agg_sh, agg_hbm, zsems)


_sc_agg_raw = pl.kernel(
    _sc_agg_body,
    out_type=jax.ShapeDtypeStruct((NC, N_NODES, D_FEAT), jnp.float32),
    mesh=_MESH,
    scratch_types=[
        [pltpu.VMEM((2, CHUNK), jnp.int32) for _ in range(NBUF)],
        [pltpu.VMEM((CHUNK, D_FEAT), jnp.float32) for _ in range(NBUF)],
        pltpu.VMEM((ZROWS, D_FEAT), jnp.float32),
        pltpu.VMEM_SHARED((N_NODES, D_FEAT), jnp.float32),
        [pltpu.SemaphoreType.DMA for _ in range(NBUF)],
        [pltpu.SemaphoreType.DMA for _ in range(NBUF)],
        [pltpu.SemaphoreType.DMA for _ in range(NBUF)],
        pltpu.SemaphoreType.DMA((2,)),
    ],
)


def _sc_deg_body(ei_hbm, deg_hbm, eis, ones_v, zbuf, deg_sh,
                 isems, ssems, zsems):
    c = lax.axis_index("c")
    s = lax.axis_index("s")
    wid = s * NC + c

    base = wid * EPW

    def fire_i(k, j):
        off = base + k * CHUNK
        pltpu.async_copy(ei_hbm.at[pl.ds(E_EDGES + off, CHUNK)],
                         eis[j].at[1], isems[j])

    def wait_i(k, j):
        off = base + k * CHUNK
        pltpu.make_async_copy(ei_hbm.at[pl.ds(E_EDGES + off, CHUNK)],
                              eis[j].at[1], isems[j]).wait()

    def fire_s(j):
        pltpu.async_copy(ones_v, deg_sh.at[eis[j].at[1]], ssems[j], add=True)

    def wait_s(j):
        pltpu.make_async_copy(ones_v, deg_sh.at[eis[j].at[1]],
                              ssems[j]).wait()

    for j in range(NBUF):
        fire_i(j, j)
    _fill(zbuf, ZROWS, jnp.zeros((L,), jnp.float32))
    _fill(ones_v, CHUNK, jnp.ones((L,), jnp.float32))
    _zero_acc(s, zbuf, deg_sh, zsems)
    plsc.subcore_barrier()

    def ring(i, _):
        for j in range(NBUF):
            k = NBUF * i + j

            @pl.when(i > 0)
            def _():
                wait_s(j)
                fire_i(k, j)

        for j in range(NBUF):
            wait_i(NBUF * i + j, j)
            fire_s(j)
        return 0

    lax.fori_loop(0, NRING // NBUF, ring, 0)
    for j in range(NBUF):
        wait_s(j)
    for k in range(NRING, NCHUNK):
        j = k - NRING
        fire_i(k, j)
        wait_i(k, j)
        fire_s(j)
        wait_s(j)

    plsc.subcore_barrier()
    _write_out(c, s, deg_sh, deg_hbm, zsems)


_sc_deg_raw = pl.kernel(
    _sc_deg_body,
    out_type=jax.ShapeDtypeStruct((NC, N_NODES, D_FEAT), jnp.float32),
    mesh=_MESH,
    scratch_types=[
        [pltpu.VMEM((2, CHUNK), jnp.int32) for _ in range(NBUF)],
        pltpu.VMEM((CHUNK, D_FEAT), jnp.float32),
        pltpu.VMEM((ZROWS, D_FEAT), jnp.float32),
        pltpu.VMEM_SHARED((N_NODES, D_FEAT), jnp.float32),
        [pltpu.SemaphoreType.DMA for _ in range(NBUF)],
        [pltpu.SemaphoreType.DMA for _ in range(NBUF)],
        pltpu.SemaphoreType.DMA((2,)),
    ],
)

ROWS_TC = 2000


def _tc_root(h, Wr, b):
    def body(h_ref, wr_ref, b_ref, o_ref):
        o_ref[...] = jnp.dot(h_ref[...], wr_ref[...],
                             preferred_element_type=jnp.float32) + b_ref[...]

    return pl.pallas_call(
        body,
        grid=(N_NODES // ROWS_TC,),
        in_specs=[
            pl.BlockSpec((ROWS_TC, D_FEAT), lambda i: (i, 0)),
            pl.BlockSpec((D_FEAT, D_FEAT), lambda i: (0, 0)),
            pl.BlockSpec((1, D_FEAT), lambda i: (0, 0)),
        ],
        out_specs=pl.BlockSpec((ROWS_TC, D_FEAT), lambda i: (i, 0)),
        out_shape=jax.ShapeDtypeStruct((N_NODES, D_FEAT), jnp.float32),
    )(h, Wr, b)


def _tc_recip(deg):
    def body(d0_ref, d1_ref, o_ref):
        o_ref[...] = 1.0 / jnp.maximum(d0_ref[0] + d1_ref[0], 1.0)

    return pl.pallas_call(
        body,
        grid=(N_NODES // ROWS_TC,),
        in_specs=[
            pl.BlockSpec((1, ROWS_TC, D_FEAT), lambda i: (0, i, 0)),
            pl.BlockSpec((1, ROWS_TC, D_FEAT), lambda i: (1, i, 0)),
        ],
        out_specs=pl.BlockSpec((ROWS_TC, D_FEAT), lambda i: (i, 0)),
        out_shape=jax.ShapeDtypeStruct((N_NODES, D_FEAT), jnp.float32),
    )(deg, deg)


def _tc_combine(hr, agg, recip, Wn, act):
    def body(hr_ref, a0_ref, a1_ref, r_ref, wn_ref, o_ref):
        mean = (a0_ref[0] + a1_ref[0]) * r_ref[...]
        acc = hr_ref[...] + jnp.dot(mean, wn_ref[...],
                                    preferred_element_type=jnp.float32)
        if act:
            acc = jnp.maximum(acc, 0.0)
        o_ref[...] = acc

    return pl.pallas_call(
        body,
        grid=(N_NODES // ROWS_TC,),
        in_specs=[
            pl.BlockSpec((ROWS_TC, D_FEAT), lambda i: (i, 0)),
            pl.BlockSpec((1, ROWS_TC, D_FEAT), lambda i: (0, i, 0)),
            pl.BlockSpec((1, ROWS_TC, D_FEAT), lambda i: (1, i, 0)),
            pl.BlockSpec((ROWS_TC, D_FEAT), lambda i: (i, 0)),
            pl.BlockSpec((D_FEAT, D_FEAT), lambda i: (0, 0)),
        ],
        out_specs=pl.BlockSpec((ROWS_TC, D_FEAT), lambda i: (i, 0)),
        out_shape=jax.ShapeDtypeStruct((N_NODES, D_FEAT), jnp.float32),
    )(hr, agg, agg, recip, Wn)


def kernel(x, edge_index, Wr0, Wn0, b0, Wr1, Wn1, b1, Wr2, Wn2, b2):
    ei = edge_index.reshape(2 * E_EDGES)
    deg = _sc_deg_raw(ei)
    agg = _sc_agg_raw(x, ei)
    recip = _tc_recip(deg)
    hr = _tc_root(x, Wr0, b0.reshape(1, D_FEAT))
    h = _tc_combine(hr, agg, recip, Wn0, True)
    agg = _sc_agg_raw(h, ei)
    hr = _tc_root(h, Wr1, b1.reshape(1, D_FEAT))
    h = _tc_combine(hr, agg, recip, Wn1, True)
    agg = _sc_agg_raw(h, ei)
    hr = _tc_root(h, Wr2, b2.reshape(1, D_FEAT))
    h = _tc_combine(hr, agg, recip, Wn2, False)
    return h


# speedup vs baseline: 10.3158x; 1.0000x over previous
import jax
import jax.numpy as jnp
from jax import lax
from jax.experimental import pallas as pl
from jax.experimental.pallas import tpu as pltpu
from jax.experimental.pallas import tpu_sc as plsc

N_NODES = 10000
D_FEAT = 128
E_EDGES = 320000

NC, NS, L = 2, 16, 16
NW = NC * NS
EPW = E_EDGES // NW
CHUNK = 80
NCHUNK = EPW // CHUNK
RPT = 624
TAIL = N_NODES - NS * RPT
ZROWS = 48

_MESH = plsc.VectorSubcoreMesh(core_axis_name="c", subcore_axis_name="s")


def _fill(buf, rows, vec):
    def row(i, _):
        for j in range(D_FEAT // L):
            buf[i, pl.ds(j * L, L)] = vec
        return 0

    lax.fori_loop(0, rows, row, 0)


def _zero_acc(s, zbuf, acc_sh, zsems):
    def zcopy(t, _):
        @pl.when(t >= 2)
        def _():
            off0 = s * RPT + (t - 2) * ZROWS
            pltpu.make_async_copy(zbuf, acc_sh.at[pl.ds(off0, ZROWS)],
                                  zsems.at[lax.rem(t, 2)]).wait()

        off = s * RPT + t * ZROWS
        pltpu.async_copy(zbuf, acc_sh.at[pl.ds(off, ZROWS)],
                         zsems.at[lax.rem(t, 2)])
        return 0

    nz = RPT // ZROWS
    lax.fori_loop(0, nz, zcopy, 0)
    for t in (nz - 2, nz - 1):
        pltpu.make_async_copy(zbuf, acc_sh.at[pl.ds(s * RPT + t * ZROWS,
                                                    ZROWS)],
                              zsems.at[t % 2]).wait()

    @pl.when(s == 0)
    def _():
        pltpu.sync_copy(zbuf.at[pl.ds(0, TAIL)],
                        acc_sh.at[pl.ds(NS * RPT, TAIL)])


def _write_out(c, s, acc_sh, out_hbm, zsems):
    pltpu.async_copy(acc_sh.at[pl.ds(s * RPT, RPT)],
                     out_hbm.at[c, pl.ds(s * RPT, RPT)], zsems.at[0])

    @pl.when(s == 0)
    def _():
        pltpu.async_copy(acc_sh.at[pl.ds(NS * RPT, TAIL)],
                         out_hbm.at[c, pl.ds(NS * RPT, TAIL)], zsems.at[1])

    pltpu.make_async_copy(acc_sh.at[pl.ds(s * RPT, RPT)],
                          out_hbm.at[c, pl.ds(s * RPT, RPT)],
                          zsems.at[0]).wait()

    @pl.when(s == 0)
    def _():
        pltpu.make_async_copy(acc_sh.at[pl.ds(NS * RPT, TAIL)],
                              out_hbm.at[c, pl.ds(NS * RPT, TAIL)],
                              zsems.at[1]).wait()


NBUF = 4
NRING = (NCHUNK // NBUF) * NBUF


def _sc_agg_body(h_hbm, ei_hbm, agg_hbm,
                 eis, rows, zbuf, agg_sh,
                 isems, gsems, ssems, zsems):
    c = lax.axis_index("c")
    s = lax.axis_index("s")
    wid = s * NC + c

    base = wid * EPW

    def fire_i(k, j):
        off = base + k * CHUNK
        pltpu.async_copy(ei_hbm.at[pl.ds(off, CHUNK)], eis[j].at[0],
                         isems[j])
        pltpu.async_copy(ei_hbm.at[pl.ds(E_EDGES + off, CHUNK)],
                         eis[j].at[1], isems[j])

    def wait_i(k, j):
        off = base + k * CHUNK
        pltpu.make_async_copy(ei_hbm.at[pl.ds(off, CHUNK)], eis[j].at[0],
                              isems[j]).wait()
        pltpu.make_async_copy(ei_hbm.at[pl.ds(E_EDGES + off, CHUNK)],
                              eis[j].at[1], isems[j]).wait()

    def fire_g(j):
        pltpu.async_copy(h_hbm.at[eis[j].at[0]], rows[j], gsems[j])

    def wait_g(j):
        pltpu.make_async_copy(h_hbm.at[eis[j].at[0]], rows[j],
                              gsems[j]).wait()

    def fire_s(j):
        pltpu.async_copy(rows[j], agg_sh.at[eis[j].at[1]], ssems[j],
                         add=True)

    def wait_s(j):
        pltpu.make_async_copy(rows[j], agg_sh.at[eis[j].at[1]],
                              ssems[j]).wait()

    for j in range(NBUF):
        fire_i(j, j)
    _fill(zbuf, ZROWS, jnp.zeros((L,), jnp.float32))
    _zero_acc(s, zbuf, agg_sh, zsems)
    plsc.subcore_barrier()

    def ring(i, _):
        for j in range(NBUF):
            k = NBUF * i + j

            @pl.when(i > 0)
            def _():
                wait_s(j)
                fire_i(k, j)

        for j in range(NBUF):
            wait_i(NBUF * i + j, j)
            fire_g(j)
        for j in range(NBUF):
            wait_g(j)
            fire_s(j)
        return 0

    lax.fori_loop(0, NRING // NBUF, ring, 0)
    for j in range(NBUF):
        wait_s(j)
    for k in range(NRING, NCHUNK):
        j = k - NRING
        fire_i(k, j)
        wait_i(k, j)
        fire_g(j)
        wait_g(j)
        fire_s(j)
        wait_s(j)

    plsc.subcore_barrier()
    _write_out(c, s, agg_sh, agg_hbm, zsems)


_sc_agg_raw = pl.kernel(
    _sc_agg_body,
    out_type=jax.ShapeDtypeStruct((NC, N_NODES, D_FEAT), jnp.float32),
    mesh=_MESH,
    scratch_types=[
        [pltpu.VMEM((2, CHUNK), jnp.int32) for _ in range(NBUF)],
        [pltpu.VMEM((CHUNK, D_FEAT), jnp.float32) for _ in range(NBUF)],
        pltpu.VMEM((ZROWS, D_FEAT), jnp.float32),
        pltpu.VMEM_SHARED((N_NODES, D_FEAT), jnp.float32),
        [pltpu.SemaphoreType.DMA for _ in range(NBUF)],
        [pltpu.SemaphoreType.DMA for _ in range(NBUF)],
        [pltpu.SemaphoreType.DMA for _ in range(NBUF)],
        pltpu.SemaphoreType.DMA((2,)),
    ],
)


def _sc_deg_body(ei_hbm, deg_hbm, eis, ones_v, zbuf, deg_sh,
                 isems, ssems, zsems):
    c = lax.axis_index("c")
    s = lax.axis_index("s")
    wid = s * NC + c

    base = wid * EPW

    def fire_i(k, j):
        off = base + k * CHUNK
        pltpu.async_copy(ei_hbm.at[pl.ds(E_EDGES + off, CHUNK)],
                         eis[j].at[1], isems[j])

    def wait_i(k, j):
        off = base + k * CHUNK
        pltpu.make_async_copy(ei_hbm.at[pl.ds(E_EDGES + off, CHUNK)],
                              eis[j].at[1], isems[j]).wait()

    def fire_s(j):
        pltpu.async_copy(ones_v, deg_sh.at[eis[j].at[1]], ssems[j], add=True)

    def wait_s(j):
        pltpu.make_async_copy(ones_v, deg_sh.at[eis[j].at[1]],
                              ssems[j]).wait()

    for j in range(NBUF):
        fire_i(j, j)
    _fill(zbuf, ZROWS, jnp.zeros((L,), jnp.float32))
    _fill(ones_v, CHUNK, jnp.ones((L,), jnp.float32))
    _zero_acc(s, zbuf, deg_sh, zsems)
    plsc.subcore_barrier()

    def ring(i, _):
        for j in range(NBUF):
            k = NBUF * i + j

            @pl.when(i > 0)
            def _():
                wait_s(j)
                fire_i(k, j)

        for j in range(NBUF):
            wait_i(NBUF * i + j, j)
            fire_s(j)
        return 0

    lax.fori_loop(0, NRING // NBUF, ring, 0)
    for j in range(NBUF):
        wait_s(j)
    for k in range(NRING, NCHUNK):
        j = k - NRING
        fire_i(k, j)
        wait_i(k, j)
        fire_s(j)
        wait_s(j)

    plsc.subcore_barrier()
    _write_out(c, s, deg_sh, deg_hbm, zsems)


_sc_deg_raw = pl.kernel(
    _sc_deg_body,
    out_type=jax.ShapeDtypeStruct((NC, N_NODES, D_FEAT), jnp.float32),
    mesh=_MESH,
    scratch_types=[
        [pltpu.VMEM((2, CHUNK), jnp.int32) for _ in range(NBUF)],
        pltpu.VMEM((CHUNK, D_FEAT), jnp.float32),
        pltpu.VMEM((ZROWS, D_FEAT), jnp.float32),
        pltpu.VMEM_SHARED((N_NODES, D_FEAT), jnp.float32),
        [pltpu.SemaphoreType.DMA for _ in range(NBUF)],
        [pltpu.SemaphoreType.DMA for _ in range(NBUF)],
        pltpu.SemaphoreType.DMA((2,)),
    ],
)

ROWS_TC = 2000


def _tc_root(h, Wr, b):
    def body(h_ref, wr_ref, b_ref, o_ref):
        o_ref[...] = jnp.dot(h_ref[...], wr_ref[...],
                             preferred_element_type=jnp.float32) + b_ref[...]

    return pl.pallas_call(
        body,
        grid=(N_NODES // ROWS_TC,),
        in_specs=[
            pl.BlockSpec((ROWS_TC, D_FEAT), lambda i: (i, 0)),
            pl.BlockSpec((D_FEAT, D_FEAT), lambda i: (0, 0)),
            pl.BlockSpec((1, D_FEAT), lambda i: (0, 0)),
        ],
        out_specs=pl.BlockSpec((ROWS_TC, D_FEAT), lambda i: (i, 0)),
        out_shape=jax.ShapeDtypeStruct((N_NODES, D_FEAT), jnp.float32),
    )(h, Wr, b)


def _tc_recip(deg):
    def body(d0_ref, d1_ref, o_ref):
        o_ref[...] = 1.0 / jnp.maximum(d0_ref[0] + d1_ref[0], 1.0)

    return pl.pallas_call(
        body,
        grid=(N_NODES // ROWS_TC,),
        in_specs=[
            pl.BlockSpec((1, ROWS_TC, D_FEAT), lambda i: (0, i, 0)),
            pl.BlockSpec((1, ROWS_TC, D_FEAT), lambda i: (1, i, 0)),
        ],
        out_specs=pl.BlockSpec((ROWS_TC, D_FEAT), lambda i: (i, 0)),
        out_shape=jax.ShapeDtypeStruct((N_NODES, D_FEAT), jnp.float32),
    )(deg, deg)


def _tc_combine(hr, agg, recip, Wn, act):
    def body(hr_ref, a0_ref, a1_ref, r_ref, wn_ref, o_ref):
        mean = (a0_ref[0] + a1_ref[0]) * r_ref[...]
        acc = hr_ref[...] + jnp.dot(mean, wn_ref[...],
                                    preferred_element_type=jnp.float32)
        if act:
            acc = jnp.maximum(acc, 0.0)
        o_ref[...] = acc

    return pl.pallas_call(
        body,
        grid=(N_NODES // ROWS_TC,),
        in_specs=[
            pl.BlockSpec((ROWS_TC, D_FEAT), lambda i: (i, 0)),
            pl.BlockSpec((1, ROWS_TC, D_FEAT), lambda i: (0, i, 0)),
            pl.BlockSpec((1, ROWS_TC, D_FEAT), lambda i: (1, i, 0)),
            pl.BlockSpec((ROWS_TC, D_FEAT), lambda i: (i, 0)),
            pl.BlockSpec((D_FEAT, D_FEAT), lambda i: (0, 0)),
        ],
        out_specs=pl.BlockSpec((ROWS_TC, D_FEAT), lambda i: (i, 0)),
        out_shape=jax.ShapeDtypeStruct((N_NODES, D_FEAT), jnp.float32),
    )(hr, agg, agg, recip, Wn)


def kernel(x, edge_index, Wr0, Wn0, b0, Wr1, Wn1, b1, Wr2, Wn2, b2):
    ei = edge_index.reshape(2 * E_EDGES)
    deg = _sc_deg_raw(ei)
    agg = _sc_agg_raw(x, ei)
    recip = _tc_recip(deg)
    hr = _tc_root(x, Wr0, b0.reshape(1, D_FEAT))
    h = _tc_combine(hr, agg, recip, Wn0, True)
    agg = _sc_agg_raw(h, ei)
    hr = _tc_root(h, Wr1, b1.reshape(1, D_FEAT))
    h = _tc_combine(hr, agg, recip, Wn1, True)
    agg = _sc_agg_raw(h, ei)
    hr = _tc_root(h, Wr2, b2.reshape(1, D_FEAT))
    h = _tc_combine(hr, agg, recip, Wn2, False)
    return h
